# Initial kernel scaffold; baseline (speedup 1.0000x reference)
import jax
import jax.numpy as jnp
from jax.experimental import pallas as pl


def kernel(f_in, edge_index, pos, max_radius, num_nodes, W1s, W2s, W1p, W2p, W1d, W2d):
    raise NotImplementedError("write your pallas kernel here")


# speedup vs baseline: 2.2408x; 2.2408x over previous
import functools
import math

import jax
import jax.numpy as jnp
import numpy as np
from jax import lax
from jax.experimental import pallas as pl
from jax.experimental.pallas import tpu as pltpu
from jax.experimental.pallas import tpu_sc as plsc

NC = 2
NS = 16
CH = 128

PD = 48
CD = 16
SD = 24


def _sel_matrix():
    srcs = ([0]
            + [(1 + u) * 9 + (1 + m) for u in range(3) for m in range(3)]
            + [(4 + u) * 9 + (4 + m) for u in range(5) for m in range(5)])
    sel = np.zeros((162, 35), np.float32)
    for j, k in enumerate(srcs):
        sel[2 * k, j] = 1.0
        sel[2 * k + 1, j] = 1.0
    return sel


def _pack_body(f_ref, pos_ref, sel_ref, p_ref, pt_ref):
    f = f_ref[...]
    bn = f.shape[0]
    pcols = jnp.dot(f, sel_ref[...], precision=lax.Precision.HIGHEST)
    posb = pos_ref[...]
    zp = jnp.zeros((bn, PD - 38), jnp.float32)
    p_ref[...] = jnp.concatenate([pcols, posb, zp], axis=1)
    pt_ref[...] = jnp.concatenate([posb, jnp.zeros((bn, CD - 3), jnp.float32)], axis=1)


def _pack(f_in, pos):
    n = f_in.shape[0]
    bn = 2000
    return pl.pallas_call(
        _pack_body,
        grid=(n // bn,),
        in_specs=[
            pl.BlockSpec((bn, 162), lambda i: (i, 0)),
            pl.BlockSpec((bn, 3), lambda i: (i, 0)),
            pl.BlockSpec((162, 35), lambda i: (0, 0)),
        ],
        out_specs=[
            pl.BlockSpec((bn, PD), lambda i: (i, 0)),
            pl.BlockSpec((bn, CD), lambda i: (i, 0)),
        ],
        out_shape=[
            jax.ShapeDtypeStruct((n, PD), jnp.float32),
            jax.ShapeDtypeStruct((n, CD), jnp.float32),
        ],
    )(f_in, pos, jnp.asarray(_sel_matrix()))


def _gather_body(p_hbm, pt_hbm, row_hbm, col_hbm, g_hbm, c_hbm,
                 idx_v, rows_v, cidx_v, crows_v, idx_t, rows_t, cidx_t, crows_t,
                 sem, e_per_w, n_full, tail):
    wid = lax.axis_index("c") * NS + lax.axis_index("s")
    base = wid * e_per_w

    def chunk(off, idxb, rowsb, cidxb, crowsb, sz):
        pltpu.sync_copy(row_hbm.at[pl.ds(off, sz)], idxb)
        pltpu.async_copy(p_hbm.at[idxb], rowsb, sem).wait()
        pltpu.sync_copy(rowsb, g_hbm.at[pl.ds(off, sz)])
        pltpu.sync_copy(col_hbm.at[pl.ds(off, sz)], cidxb)
        pltpu.async_copy(pt_hbm.at[cidxb], crowsb, sem).wait()
        pltpu.sync_copy(crowsb, c_hbm.at[pl.ds(off, sz)])

    def body(i, carry):
        chunk(base + i * CH, idx_v, rows_v, cidx_v, crows_v, CH)
        return carry

    lax.fori_loop(0, n_full, body, 0)
    if tail:
        chunk(base + n_full * CH, idx_t, rows_t, cidx_t, crows_t, tail)


def _gather(P, PT, row, col):
    n = P.shape[0]
    e = row.shape[0]
    nw = NC * NS
    e_per_w = e // nw
    assert e_per_w * nw == e and e_per_w % 8 == 0
    n_full = e_per_w // CH
    tail = e_per_w - n_full * CH
    assert tail % 8 == 0

    mesh = plsc.VectorSubcoreMesh(core_axis_name="c", subcore_axis_name="s")
    scratch = [
        pltpu.VMEM((CH,), jnp.int32),
        pltpu.VMEM((CH, PD), jnp.float32),
        pltpu.VMEM((CH,), jnp.int32),
        pltpu.VMEM((CH, CD), jnp.float32),
        pltpu.VMEM((max(tail, 8),), jnp.int32),
        pltpu.VMEM((max(tail, 8), PD), jnp.float32),
        pltpu.VMEM((max(tail, 8),), jnp.int32),
        pltpu.VMEM((max(tail, 8), CD), jnp.float32),
        pltpu.SemaphoreType.DMA,
    ]
    kfn = functools.partial(
        pl.kernel,
        mesh=mesh,
        out_type=[
            jax.ShapeDtypeStruct((e, PD), jnp.float32),
            jax.ShapeDtypeStruct((e, CD), jnp.float32),
        ],
        scratch_types=scratch,
        compiler_params=pltpu.CompilerParams(use_tc_tiling_on_sc=False),
    )(functools.partial(_gather_body, e_per_w=e_per_w, n_full=n_full, tail=tail))
    return kfn(P, PT, row, col)


def _dense_body(g_ref, c_ref, vals_ref, istep_ref, a1_ref, a2s_ref, a2p_ref, a2d_ref,
                s0_ref, s1_ref):
    g = g_ref[...]
    c = c_ref[...]
    be = g.shape[0]
    v = g[:, 35:38] - c[:, 0:3]
    ln = jnp.sqrt(jnp.sum(v * v, axis=1, keepdims=True) + 1e-12)
    r = v / ln
    x, y, z = r[:, 0:1], r[:, 1:2], r[:, 2:3]
    s3 = math.sqrt(3.0)
    s15 = math.sqrt(15.0)
    sh1 = (s3 * x, s3 * y, s3 * z)
    sh2 = ((s15) * x * y, s15 * y * z, (math.sqrt(5.0) / 2.0) * (3.0 * z * z - 1.0),
           s15 * x * z, (s15 / 2.0) * (x * x - y * y))

    diff = (ln - vals_ref[...]) * istep_ref[...]
    def sus(t):
        safe = jnp.where(t > 0.0, t, 1.0)
        return jnp.where(t > 0.0, jnp.exp(-1.0 / safe), 0.0)
    cemb = 1.14136 * math.exp(2.0) * math.sqrt(10.0)
    emb = cemb * sus(diff + 1.0) * sus(1.0 - diff)

    h = jnp.maximum(jnp.dot(emb, a1_ref[...]), 0.0)
    ws = jnp.dot(h[:, 0:64], a2s_ref[...])
    wp = jnp.dot(h[:, 64:128], a2p_ref[...])
    wd = jnp.dot(h[:, 128:192], a2d_ref[...])

    out_s = g[:, 0:1] * ws
    out_p = jnp.zeros((be, 16), jnp.float32)
    for u in range(3):
        dp = (g[:, 1 + 3 * u:2 + 3 * u] * sh1[0]
              + g[:, 2 + 3 * u:3 + 3 * u] * sh1[1]
              + g[:, 3 + 3 * u:4 + 3 * u] * sh1[2])
        out_p = out_p + dp * wp[:, 16 * u:16 * u + 16]
    out_d = jnp.zeros((be, 16), jnp.float32)
    for u in range(5):
        dd = jnp.zeros((be, 1), jnp.float32)
        for m in range(5):
            dd = dd + g[:, 10 + 5 * u + m:11 + 5 * u + m] * sh2[m]
        out_d = out_d + dd * wd[:, 16 * u:16 * u + 16]

    s0_ref[...] = jnp.concatenate([out_s, out_p[:, 0:8]], axis=1)
    s1_ref[...] = jnp.concatenate([out_p[:, 8:16], out_d], axis=1)


def _dense(G, C, vals, istep, a1, a2s, a2p, a2d):
    e = G.shape[0]
    be = 1600
    return pl.pallas_call(
        _dense_body,
        grid=(e // be,),
        in_specs=[
            pl.BlockSpec((be, PD), lambda i: (i, 0)),
            pl.BlockSpec((be, CD), lambda i: (i, 0)),
            pl.BlockSpec((1, 10), lambda i: (0, 0)),
            pl.BlockSpec((1, 1), lambda i: (0, 0)),
            pl.BlockSpec((10, 192), lambda i: (0, 0)),
            pl.BlockSpec((64, 16), lambda i: (0, 0)),
            pl.BlockSpec((64, 48), lambda i: (0, 0)),
            pl.BlockSpec((64, 80), lambda i: (0, 0)),
        ],
        out_specs=[
            pl.BlockSpec((be, SD), lambda i: (i, 0)),
            pl.BlockSpec((be, SD), lambda i: (i, 0)),
        ],
        out_shape=[
            jax.ShapeDtypeStruct((e, SD), jnp.float32),
            jax.ShapeDtypeStruct((e, SD), jnp.float32),
        ],
    )(G, C, vals, istep, a1, a2s, a2p, a2d)


def _scatter_body(s0_hbm, s1_hbm, col_hbm, z_hbm, o_hbm,
                  acc, idx_v, rows_v, idx_t, rows_t,
                  e_per_s, n_full, tail, n_per_s):
    c = lax.axis_index("c")
    s = lax.axis_index("s")
    pltpu.sync_copy(z_hbm, acc.at[pl.ds(s * n_per_s, n_per_s)])
    plsc.subcore_barrier()

    base = s * e_per_s

    def run(s_hbm):
        def chunk(off, idxb, rowsb, sz):
            pltpu.sync_copy(col_hbm.at[pl.ds(off, sz)], idxb)
            pltpu.sync_copy(s_hbm.at[pl.ds(off, sz)], rowsb)
            pltpu.sync_copy(rowsb, acc.at[idxb], add=True)

        def body(i, carry):
            chunk(base + i * CH, idx_v, rows_v, CH)
            return carry

        lax.fori_loop(0, n_full, body, 0)
        if tail:
            chunk(base + n_full * CH, idx_t, rows_t, tail)

    @pl.when(c == 0)
    def _():
        run(s0_hbm)

    @pl.when(c == 1)
    def _():
        run(s1_hbm)

    plsc.subcore_barrier()
    pltpu.sync_copy(acc.at[pl.ds(s * n_per_s, n_per_s)],
                    o_hbm.at[c, pl.ds(s * n_per_s, n_per_s)])


def _scatter(S0, S1, col, n):
    e = S0.shape[0]
    e_per_s = e // NS
    assert e_per_s * NS == e and e_per_s % 8 == 0
    n_full = e_per_s // CH
    tail = e_per_s - n_full * CH
    assert tail % 8 == 0
    n_per_s = n // NS
    assert n_per_s * NS == n

    z = jnp.zeros((n_per_s, SD), jnp.float32)
    mesh = plsc.VectorSubcoreMesh(core_axis_name="c", subcore_axis_name="s")
    scratch = [
        pltpu.VMEM_SHARED((n, SD), jnp.float32),
        pltpu.VMEM((CH,), jnp.int32),
        pltpu.VMEM((CH, SD), jnp.float32),
        pltpu.VMEM((max(tail, 8),), jnp.int32),
        pltpu.VMEM((max(tail, 8), SD), jnp.float32),
    ]
    kfn = functools.partial(
        pl.kernel,
        mesh=mesh,
        out_type=jax.ShapeDtypeStruct((NC, n, SD), jnp.float32),
        scratch_types=scratch,
        compiler_params=pltpu.CompilerParams(use_tc_tiling_on_sc=False),
    )(functools.partial(_scatter_body, e_per_s=e_per_s, n_full=n_full,
                        tail=tail, n_per_s=n_per_s))
    return kfn(S0, S1, col, z)


def kernel(f_in, edge_index, pos, max_radius, num_nodes, W1s, W2s, W1p, W2p, W1d, W2d):
    n = f_in.shape[0]
    e = edge_index.shape[1]
    row = edge_index[0]
    col = edge_index[1]

    num_neighbors = e / n
    a1 = jnp.concatenate([W1s, W1p, W1d], axis=1) * (1.0 / math.sqrt(10.0))
    cs = math.sqrt(2.0) / 8.0 / math.sqrt(num_neighbors)
    a2s = W2s * cs
    a2p = W2p * (cs / 3.0)
    a2d = W2d * (cs / 5.0)

    mr = jnp.asarray(max_radius, jnp.float32)
    step = mr / (10 + 1)
    vals = (jnp.arange(1, 11, dtype=jnp.float32) * step).reshape(1, 10)
    istep = (1.0 / step).reshape(1, 1)

    P, PT = _pack(f_in, pos)
    G, C = _gather(P, PT, row, col)
    S0, S1 = _dense(G, C, vals, istep, a1, a2s, a2p, a2d)
    O = _scatter(S0, S1, col, n)
    return jnp.concatenate([O[0], O[1]], axis=1)


# speedup vs baseline: 6.8117x; 3.0399x over previous
import functools
import math

import jax
import jax.numpy as jnp
import numpy as np
from jax import lax
from jax.experimental import pallas as pl
from jax.experimental.pallas import tpu as pltpu
from jax.experimental.pallas import tpu_sc as plsc

NC = 2
NS = 16
CH = 128

PD = 48
CD = 16
SD = 24


def _sel_matrix():
    srcs = ([0]
            + [(1 + u) * 9 + (1 + m) for u in range(3) for m in range(3)]
            + [(4 + u) * 9 + (4 + m) for u in range(5) for m in range(5)])
    sel = np.zeros((162, 35), np.float32)
    for j, k in enumerate(srcs):
        sel[2 * k, j] = 1.0
        sel[2 * k + 1, j] = 1.0
    return sel


def _pack_body(f_ref, pos_ref, sel_ref, p_ref, pt_ref):
    f = f_ref[...]
    bn = f.shape[0]
    pcols = jnp.dot(f, sel_ref[...], precision=lax.Precision.HIGHEST)
    posb = pos_ref[...]
    zp = jnp.zeros((bn, PD - 38), jnp.float32)
    p_ref[...] = jnp.concatenate([pcols, posb, zp], axis=1)
    pt_ref[...] = jnp.concatenate([posb, jnp.zeros((bn, CD - 3), jnp.float32)], axis=1)


def _pack(f_in, pos):
    n = f_in.shape[0]
    bn = 2000
    return pl.pallas_call(
        _pack_body,
        grid=(n // bn,),
        in_specs=[
            pl.BlockSpec((bn, 162), lambda i: (i, 0)),
            pl.BlockSpec((bn, 3), lambda i: (i, 0)),
            pl.BlockSpec((162, 35), lambda i: (0, 0)),
        ],
        out_specs=[
            pl.BlockSpec((bn, PD), lambda i: (i, 0)),
            pl.BlockSpec((bn, CD), lambda i: (i, 0)),
        ],
        out_shape=[
            jax.ShapeDtypeStruct((n, PD), jnp.float32),
            jax.ShapeDtypeStruct((n, CD), jnp.float32),
        ],
    )(f_in, pos, jnp.asarray(_sel_matrix()))


def _gather_body(p_hbm, pt_hbm, row_hbm, col_hbm, g_hbm, c_hbm,
                 idx_v, rows_v, cidx_v, crows_v, idx_t, rows_t, cidx_t, crows_t,
                 sem, e_per_w, n_full, tail):
    wid = lax.axis_index("c") * NS + lax.axis_index("s")
    base = wid * e_per_w

    def chunk(off, idxb, rowsb, cidxb, crowsb, sz):
        pltpu.sync_copy(row_hbm.at[pl.ds(off, sz)], idxb)
        pltpu.async_copy(p_hbm.at[idxb], rowsb, sem).wait()
        pltpu.sync_copy(rowsb, g_hbm.at[pl.ds(off, sz)])
        pltpu.sync_copy(col_hbm.at[pl.ds(off, sz)], cidxb)
        pltpu.async_copy(pt_hbm.at[cidxb], crowsb, sem).wait()
        pltpu.sync_copy(crowsb, c_hbm.at[pl.ds(off, sz)])

    def body(i, carry):
        chunk(base + i * CH, idx_v, rows_v, cidx_v, crows_v, CH)
        return carry

    lax.fori_loop(0, n_full, body, 0)
    if tail:
        chunk(base + n_full * CH, idx_t, rows_t, cidx_t, crows_t, tail)


def _gather(P, PT, row, col):
    n = P.shape[0]
    e = row.shape[0]
    nw = NC * NS
    e_per_w = e // nw
    assert e_per_w * nw == e and e_per_w % 8 == 0
    n_full = e_per_w // CH
    tail = e_per_w - n_full * CH
    assert tail % 8 == 0

    mesh = plsc.VectorSubcoreMesh(core_axis_name="c", subcore_axis_name="s")
    scratch = [
        pltpu.VMEM((CH,), jnp.int32),
        pltpu.VMEM((CH, PD), jnp.float32),
        pltpu.VMEM((CH,), jnp.int32),
        pltpu.VMEM((CH, CD), jnp.float32),
        pltpu.VMEM((max(tail, 8),), jnp.int32),
        pltpu.VMEM((max(tail, 8), PD), jnp.float32),
        pltpu.VMEM((max(tail, 8),), jnp.int32),
        pltpu.VMEM((max(tail, 8), CD), jnp.float32),
        pltpu.SemaphoreType.DMA,
    ]
    kfn = functools.partial(
        pl.kernel,
        mesh=mesh,
        out_type=[
            jax.ShapeDtypeStruct((e, PD), jnp.float32),
            jax.ShapeDtypeStruct((e, CD), jnp.float32),
        ],
        scratch_types=scratch,
        compiler_params=pltpu.CompilerParams(use_tc_tiling_on_sc=False),
    )(functools.partial(_gather_body, e_per_w=e_per_w, n_full=n_full, tail=tail))
    return kfn(P, PT, row, col)


def _dense_body(g_ref, c_ref, vals_ref, istep_ref, a1_ref, a2s_ref, a2p_ref, a2d_ref,
                s0_ref, s1_ref):
    be = g_ref.shape[0]
    gt = g_ref[...].T
    ct = c_ref[...].T

    v = gt[35:38, :] - ct[0:3, :]
    n2 = jnp.sum(v * v, axis=0, keepdims=True) + 1e-12
    ln = jnp.sqrt(n2)
    inv = 1.0 / ln
    x, y, z = v[0:1, :] * inv, v[1:2, :] * inv, v[2:3, :] * inv
    s3 = math.sqrt(3.0)
    s15 = math.sqrt(15.0)
    sh1 = (s3 * x, s3 * y, s3 * z)
    sh2 = (s15 * x * y, s15 * y * z, (math.sqrt(5.0) / 2.0) * (3.0 * z * z - 1.0),
           s15 * x * z, (s15 / 2.0) * (x * x - y * y))

    lnb = jnp.broadcast_to(ln, (10, be))
    diff = (lnb - vals_ref[...]) * istep_ref[...]
    def sus(t):
        safe = jnp.where(t > 0.0, t, 1.0)
        return jnp.where(t > 0.0, jnp.exp(-1.0 / safe), 0.0)
    cemb = 1.14136 * math.exp(2.0) * math.sqrt(10.0)
    emb = cemb * sus(diff + 1.0) * sus(1.0 - diff)

    h = jnp.maximum(jnp.dot(a1_ref[...], emb), 0.0)
    ws = jnp.dot(a2s_ref[...], h[0:64, :])
    wp = jnp.dot(a2p_ref[...], h[64:128, :])
    wd = jnp.dot(a2d_ref[...], h[128:192, :])

    out_s = jnp.broadcast_to(gt[0:1, :], (16, be)) * ws
    out_p = jnp.zeros((16, be), jnp.float32)
    for u in range(3):
        dp = (gt[1 + 3 * u:2 + 3 * u, :] * sh1[0]
              + gt[2 + 3 * u:3 + 3 * u, :] * sh1[1]
              + gt[3 + 3 * u:4 + 3 * u, :] * sh1[2])
        out_p = out_p + jnp.broadcast_to(dp, (16, be)) * wp[16 * u:16 * u + 16, :]
    out_d = jnp.zeros((16, be), jnp.float32)
    for u in range(5):
        dd = jnp.zeros((1, be), jnp.float32)
        for m in range(5):
            dd = dd + gt[10 + 5 * u + m:11 + 5 * u + m, :] * sh2[m]
        out_d = out_d + jnp.broadcast_to(dd, (16, be)) * wd[16 * u:16 * u + 16, :]

    outt = jnp.concatenate([out_s, out_p, out_d], axis=0)
    out = outt.T
    s0_ref[...] = out[:, 0:SD]
    s1_ref[...] = out[:, SD:2 * SD]


def _dense(G, C, vals, istep, a1, a2s, a2p, a2d):
    e = G.shape[0]
    be = 1600
    return pl.pallas_call(
        _dense_body,
        grid=(e // be,),
        in_specs=[
            pl.BlockSpec((be, PD), lambda i: (i, 0)),
            pl.BlockSpec((be, CD), lambda i: (i, 0)),
            pl.BlockSpec((10, 1), lambda i: (0, 0)),
            pl.BlockSpec((1, 1), lambda i: (0, 0)),
            pl.BlockSpec((192, 10), lambda i: (0, 0)),
            pl.BlockSpec((16, 64), lambda i: (0, 0)),
            pl.BlockSpec((48, 64), lambda i: (0, 0)),
            pl.BlockSpec((80, 64), lambda i: (0, 0)),
        ],
        out_specs=[
            pl.BlockSpec((be, SD), lambda i: (i, 0)),
            pl.BlockSpec((be, SD), lambda i: (i, 0)),
        ],
        out_shape=[
            jax.ShapeDtypeStruct((e, SD), jnp.float32),
            jax.ShapeDtypeStruct((e, SD), jnp.float32),
        ],
    )(G, C, vals, istep, a1, a2s, a2p, a2d)


def _scatter_body(s0_hbm, s1_hbm, col_hbm, z_hbm, o_hbm,
                  acc, idx_v, rows_v, idx_t, rows_t,
                  e_per_s, n_full, tail, n_per_s):
    c = lax.axis_index("c")
    s = lax.axis_index("s")
    pltpu.sync_copy(z_hbm, acc.at[pl.ds(s * n_per_s, n_per_s)])
    plsc.subcore_barrier()

    base = s * e_per_s

    def run(s_hbm):
        def chunk(off, idxb, rowsb, sz):
            pltpu.sync_copy(col_hbm.at[pl.ds(off, sz)], idxb)
            pltpu.sync_copy(s_hbm.at[pl.ds(off, sz)], rowsb)
            pltpu.sync_copy(rowsb, acc.at[idxb], add=True)

        def body(i, carry):
            chunk(base + i * CH, idx_v, rows_v, CH)
            return carry

        lax.fori_loop(0, n_full, body, 0)
        if tail:
            chunk(base + n_full * CH, idx_t, rows_t, tail)

    @pl.when(c == 0)
    def _():
        run(s0_hbm)

    @pl.when(c == 1)
    def _():
        run(s1_hbm)

    plsc.subcore_barrier()
    pltpu.sync_copy(acc.at[pl.ds(s * n_per_s, n_per_s)],
                    o_hbm.at[c, pl.ds(s * n_per_s, n_per_s)])


def _scatter(S0, S1, col, n):
    e = S0.shape[0]
    e_per_s = e // NS
    assert e_per_s * NS == e and e_per_s % 8 == 0
    n_full = e_per_s // CH
    tail = e_per_s - n_full * CH
    assert tail % 8 == 0
    n_per_s = n // NS
    assert n_per_s * NS == n

    z = jnp.zeros((n_per_s, SD), jnp.float32)
    mesh = plsc.VectorSubcoreMesh(core_axis_name="c", subcore_axis_name="s")
    scratch = [
        pltpu.VMEM_SHARED((n, SD), jnp.float32),
        pltpu.VMEM((CH,), jnp.int32),
        pltpu.VMEM((CH, SD), jnp.float32),
        pltpu.VMEM((max(tail, 8),), jnp.int32),
        pltpu.VMEM((max(tail, 8), SD), jnp.float32),
    ]
    kfn = functools.partial(
        pl.kernel,
        mesh=mesh,
        out_type=jax.ShapeDtypeStruct((NC, n, SD), jnp.float32),
        scratch_types=scratch,
        compiler_params=pltpu.CompilerParams(use_tc_tiling_on_sc=False),
    )(functools.partial(_scatter_body, e_per_s=e_per_s, n_full=n_full,
                        tail=tail, n_per_s=n_per_s))
    return kfn(S0, S1, col, z)


def kernel(f_in, edge_index, pos, max_radius, num_nodes, W1s, W2s, W1p, W2p, W1d, W2d):
    n = f_in.shape[0]
    e = edge_index.shape[1]
    row = edge_index[0]
    col = edge_index[1]

    num_neighbors = e / n
    a1 = (jnp.concatenate([W1s, W1p, W1d], axis=1) * (1.0 / math.sqrt(10.0))).T
    cs = math.sqrt(2.0) / 8.0 / math.sqrt(num_neighbors)
    a2s = (W2s * cs).T
    a2p = (W2p * (cs / 3.0)).T
    a2d = (W2d * (cs / 5.0)).T

    mr = jnp.asarray(max_radius, jnp.float32)
    step = mr / (10 + 1)
    vals = (jnp.arange(1, 11, dtype=jnp.float32) * step).reshape(10, 1)
    istep = (1.0 / step).reshape(1, 1)

    P, PT = _pack(f_in, pos)
    G, C = _gather(P, PT, row, col)
    S0, S1 = _dense(G, C, vals, istep, a1, a2s, a2p, a2d)
    O = _scatter(S0, S1, col, n)
    return jnp.concatenate([O[0], O[1]], axis=1)


# speedup vs baseline: 11.5198x; 1.6912x over previous
import functools
import math

import jax
import jax.numpy as jnp
import numpy as np
from jax import lax
from jax.experimental import pallas as pl
from jax.experimental.pallas import tpu as pltpu
from jax.experimental.pallas import tpu_sc as plsc

NC = 2
NS = 16
CH = 128

PD = 48
CD = 16
SD = 24


def _sel_matrix():
    srcs = ([0]
            + [(1 + u) * 9 + (1 + m) for u in range(3) for m in range(3)]
            + [(4 + u) * 9 + (4 + m) for u in range(5) for m in range(5)])
    sel = np.zeros((162, 35), np.float32)
    for j, k in enumerate(srcs):
        sel[2 * k, j] = 1.0
        sel[2 * k + 1, j] = 1.0
    return sel


def _pack_body(f_ref, pos_ref, sel_ref, p_ref, pt_ref):
    f = f_ref[...]
    bn = f.shape[0]
    pcols = jnp.dot(f, sel_ref[...], precision=lax.Precision.HIGHEST)
    posb = pos_ref[...]
    zp = jnp.zeros((bn, PD - 38), jnp.float32)
    p_ref[...] = jnp.concatenate([pcols, posb, zp], axis=1)
    pt_ref[...] = jnp.concatenate([posb, jnp.zeros((bn, CD - 3), jnp.float32)], axis=1)


def _pack(f_in, pos):
    n = f_in.shape[0]
    bn = 2000
    return pl.pallas_call(
        _pack_body,
        grid=(n // bn,),
        in_specs=[
            pl.BlockSpec((bn, 162), lambda i: (i, 0)),
            pl.BlockSpec((bn, 3), lambda i: (i, 0)),
            pl.BlockSpec((162, 35), lambda i: (0, 0)),
        ],
        out_specs=[
            pl.BlockSpec((bn, PD), lambda i: (i, 0)),
            pl.BlockSpec((bn, CD), lambda i: (i, 0)),
        ],
        out_shape=[
            jax.ShapeDtypeStruct((n, PD), jnp.float32),
            jax.ShapeDtypeStruct((n, CD), jnp.float32),
        ],
    )(f_in, pos, jnp.asarray(_sel_matrix()))


def _gather_body(p_hbm, pt_hbm, row_hbm, col_hbm, g_hbm,
                 idx_v, rows_v, cidx_v, crows_v, idx_t, rows_t, cidx_t, crows_t,
                 sem, e_per_w, n_full, tail):
    wid = lax.axis_index("c") * NS + lax.axis_index("s")
    base = wid * e_per_w

    def chunk(off, idxb, rowsb, cidxb, crowsb, sz):
        pltpu.sync_copy(row_hbm.at[pl.ds(off, sz)], idxb)
        pltpu.async_copy(p_hbm.at[idxb], rowsb, sem).wait()
        pltpu.sync_copy(rowsb, g_hbm.at[pl.ds(off, sz), pl.ds(0, PD)])
        pltpu.sync_copy(col_hbm.at[pl.ds(off, sz)], cidxb)
        pltpu.async_copy(pt_hbm.at[cidxb], crowsb, sem).wait()
        pltpu.sync_copy(crowsb, g_hbm.at[pl.ds(off, sz), pl.ds(PD, CD)])

    def body(i, carry):
        chunk(base + i * CH, idx_v, rows_v, cidx_v, crows_v, CH)
        return carry

    lax.fori_loop(0, n_full, body, 0)
    if tail:
        chunk(base + n_full * CH, idx_t, rows_t, cidx_t, crows_t, tail)


def _gather(P, PT, row, col):
    e = row.shape[0]
    nw = NC * NS
    e_per_w = e // nw
    assert e_per_w * nw == e and e_per_w % 8 == 0
    n_full = e_per_w // CH
    tail = e_per_w - n_full * CH
    assert tail % 8 == 0

    mesh = plsc.VectorSubcoreMesh(core_axis_name="c", subcore_axis_name="s")
    scratch = [
        pltpu.VMEM((CH,), jnp.int32),
        pltpu.VMEM((CH, PD), jnp.float32),
        pltpu.VMEM((CH,), jnp.int32),
        pltpu.VMEM((CH, CD), jnp.float32),
        pltpu.VMEM((max(tail, 8),), jnp.int32),
        pltpu.VMEM((max(tail, 8), PD), jnp.float32),
        pltpu.VMEM((max(tail, 8),), jnp.int32),
        pltpu.VMEM((max(tail, 8), CD), jnp.float32),
        pltpu.SemaphoreType.DMA,
    ]
    kfn = functools.partial(
        pl.kernel,
        mesh=mesh,
        out_type=jax.ShapeDtypeStruct((e, 128), jnp.float32),
        scratch_types=scratch,
        compiler_params=pltpu.CompilerParams(use_tc_tiling_on_sc=False),
    )(functools.partial(_gather_body, e_per_w=e_per_w, n_full=n_full, tail=tail))
    return kfn(P, PT, row, col)


def _dense_body(g_ref, vals_ref, istep_ref, a1_ref, a2s_ref, a2p_ref, a2d_ref,
                s_ref):
    be = g_ref.shape[0]
    gt = g_ref[:, 0:64].T

    v = gt[35:38, :] - gt[PD:PD + 3, :]
    n2 = jnp.sum(v * v, axis=0, keepdims=True) + 1e-12
    ln = jnp.sqrt(n2)
    inv = 1.0 / ln
    x, y, z = v[0:1, :] * inv, v[1:2, :] * inv, v[2:3, :] * inv
    s3 = math.sqrt(3.0)
    s15 = math.sqrt(15.0)
    sh1 = (s3 * x, s3 * y, s3 * z)
    sh2 = (s15 * x * y, s15 * y * z, (math.sqrt(5.0) / 2.0) * (3.0 * z * z - 1.0),
           s15 * x * z, (s15 / 2.0) * (x * x - y * y))

    lnb = jnp.broadcast_to(ln, (10, be))
    diff = (lnb - vals_ref[...]) * istep_ref[...]
    def sus(t):
        safe = jnp.where(t > 0.0, t, 1.0)
        return jnp.where(t > 0.0, jnp.exp(-1.0 / safe), 0.0)
    cemb = 1.14136 * math.exp(2.0) * math.sqrt(10.0)
    emb = cemb * sus(diff + 1.0) * sus(1.0 - diff)

    h = jnp.maximum(jnp.dot(a1_ref[...], emb), 0.0)
    ws = jnp.dot(a2s_ref[...], h[0:64, :])
    wp = jnp.dot(a2p_ref[...], h[64:128, :])
    wd = jnp.dot(a2d_ref[...], h[128:192, :])

    out_s = jnp.broadcast_to(gt[0:1, :], (16, be)) * ws
    out_p = jnp.zeros((16, be), jnp.float32)
    for u in range(3):
        dp = (gt[1 + 3 * u:2 + 3 * u, :] * sh1[0]
              + gt[2 + 3 * u:3 + 3 * u, :] * sh1[1]
              + gt[3 + 3 * u:4 + 3 * u, :] * sh1[2])
        out_p = out_p + jnp.broadcast_to(dp, (16, be)) * wp[16 * u:16 * u + 16, :]
    out_d = jnp.zeros((16, be), jnp.float32)
    for u in range(5):
        dd = jnp.zeros((1, be), jnp.float32)
        for m in range(5):
            dd = dd + gt[10 + 5 * u + m:11 + 5 * u + m, :] * sh2[m]
        out_d = out_d + jnp.broadcast_to(dd, (16, be)) * wd[16 * u:16 * u + 16, :]

    outt = jnp.concatenate([out_s, out_p, out_d], axis=0)
    out = outt.T
    s_ref[...] = jnp.concatenate(
        [out, jnp.zeros((be, 128 - 2 * SD), jnp.float32)], axis=1)


def _dense(G, vals, istep, a1, a2s, a2p, a2d):
    e = G.shape[0]
    be = 1600
    return pl.pallas_call(
        _dense_body,
        grid=(e // be,),
        in_specs=[
            pl.BlockSpec((be, 128), lambda i: (i, 0)),
            pl.BlockSpec((10, 1), lambda i: (0, 0)),
            pl.BlockSpec((1, 1), lambda i: (0, 0)),
            pl.BlockSpec((192, 10), lambda i: (0, 0)),
            pl.BlockSpec((16, 64), lambda i: (0, 0)),
            pl.BlockSpec((48, 64), lambda i: (0, 0)),
            pl.BlockSpec((80, 64), lambda i: (0, 0)),
        ],
        out_specs=[
            pl.BlockSpec((be, 128), lambda i: (i, 0)),
        ],
        out_shape=[
            jax.ShapeDtypeStruct((e, 128), jnp.float32),
        ],
    )(G, vals, istep, a1, a2s, a2p, a2d)[0]


def _scatter_body(s_hbm, col_hbm, z_hbm, o_hbm,
                  acc, idx_v, rows_v, idx_t, rows_t,
                  e_per_s, n_full, tail, n_per_s, n):
    c = lax.axis_index("c")
    s = lax.axis_index("s")
    r0 = s * n_per_s
    n_last = n - (NS - 1) * n_per_s

    @pl.when(s < NS - 1)
    def _():
        pltpu.sync_copy(z_hbm.at[pl.ds(0, n_per_s)], acc.at[pl.ds(r0, n_per_s)])

    @pl.when(s == NS - 1)
    def _():
        pltpu.sync_copy(z_hbm.at[pl.ds(0, n_last)], acc.at[pl.ds(r0, n_last)])

    plsc.subcore_barrier()

    base = s * e_per_s
    coff = c * SD

    def chunk(off, idxb, rowsb, sz):
        pltpu.sync_copy(col_hbm.at[pl.ds(off, sz)], idxb)
        pltpu.sync_copy(s_hbm.at[pl.ds(off, sz), pl.ds(coff, SD)], rowsb)
        pltpu.sync_copy(rowsb, acc.at[idxb], add=True)

    def body(i, carry):
        chunk(base + i * CH, idx_v, rows_v, CH)
        return carry

    lax.fori_loop(0, n_full, body, 0)
    if tail:
        chunk(base + n_full * CH, idx_t, rows_t, tail)

    plsc.subcore_barrier()

    @pl.when(s < NS - 1)
    def _():
        pltpu.sync_copy(acc.at[pl.ds(r0, n_per_s)],
                        o_hbm.at[pl.ds(r0, n_per_s), pl.ds(coff, SD)])

    @pl.when(s == NS - 1)
    def _():
        pltpu.sync_copy(acc.at[pl.ds(r0, n_last)],
                        o_hbm.at[pl.ds(r0, n_last), pl.ds(coff, SD)])


def _scatter(S, col, n):
    e = S.shape[0]
    e_per_s = e // NS
    assert e_per_s * NS == e and e_per_s % 8 == 0
    n_full = e_per_s // CH
    tail = e_per_s - n_full * CH
    assert tail % 8 == 0
    n_per_s = -(-n // NS)
    n_per_s += (-n_per_s) % 8
    assert (NS - 1) * n_per_s < n

    z = jnp.zeros((n_per_s, SD), jnp.float32)
    mesh = plsc.VectorSubcoreMesh(core_axis_name="c", subcore_axis_name="s")
    scratch = [
        pltpu.VMEM_SHARED((n, SD), jnp.float32),
        pltpu.VMEM((CH,), jnp.int32),
        pltpu.VMEM((CH, SD), jnp.float32),
        pltpu.VMEM((max(tail, 8),), jnp.int32),
        pltpu.VMEM((max(tail, 8), SD), jnp.float32),
    ]
    kfn = functools.partial(
        pl.kernel,
        mesh=mesh,
        out_type=jax.ShapeDtypeStruct((n, 128), jnp.float32),
        scratch_types=scratch,
        compiler_params=pltpu.CompilerParams(use_tc_tiling_on_sc=False),
    )(functools.partial(_scatter_body, e_per_s=e_per_s, n_full=n_full,
                        tail=tail, n_per_s=n_per_s, n=n))
    return kfn(S, col, z)


def kernel(f_in, edge_index, pos, max_radius, num_nodes, W1s, W2s, W1p, W2p, W1d, W2d):
    n = f_in.shape[0]
    e = edge_index.shape[1]
    row = edge_index[0]
    col = edge_index[1]

    num_neighbors = e / n
    a1 = (jnp.concatenate([W1s, W1p, W1d], axis=1) * (1.0 / math.sqrt(10.0))).T
    cs = math.sqrt(2.0) / 8.0 / math.sqrt(num_neighbors)
    a2s = (W2s * cs).T
    a2p = (W2p * (cs / 3.0)).T
    a2d = (W2d * (cs / 5.0)).T

    mr = jnp.asarray(max_radius, jnp.float32)
    step = mr / (10 + 1)
    vals = (jnp.arange(1, 11, dtype=jnp.float32) * step).reshape(10, 1)
    istep = (1.0 / step).reshape(1, 1)

    P, PT = _pack(f_in, pos)
    G = _gather(P, PT, row, col)
    S = _dense(G, vals, istep, a1, a2s, a2p, a2d)
    O = _scatter(S, col, n)
    return O[:, 0:2 * SD]


# speedup vs baseline: 19.0053x; 1.6498x over previous
import functools
import math

import jax
import jax.numpy as jnp
import numpy as np
from jax import lax
from jax.experimental import pallas as pl
from jax.experimental.pallas import tpu as pltpu
from jax.experimental.pallas import tpu_sc as plsc

NC = 2
NS = 16
CH = 128

PD = 48
CD = 16
SD = 24


def _sel_matrix():
    srcs = ([0]
            + [(1 + u) * 9 + (1 + m) for u in range(3) for m in range(3)]
            + [(4 + u) * 9 + (4 + m) for u in range(5) for m in range(5)])
    sel = np.zeros((162, 35), np.float32)
    for j, k in enumerate(srcs):
        sel[2 * k, j] = 1.0
        sel[2 * k + 1, j] = 1.0
    return sel


def _pack_body(f_ref, pos_ref, sel_ref, p_ref, pt_ref):
    f = f_ref[...]
    bn = f.shape[0]
    pcols = jnp.dot(f, sel_ref[...], precision=lax.Precision.HIGHEST)
    posb = pos_ref[...]
    zp = jnp.zeros((bn, PD - 38), jnp.float32)
    p_ref[...] = jnp.concatenate([pcols, posb, zp], axis=1)
    pt_ref[...] = jnp.concatenate([posb, jnp.zeros((bn, CD - 3), jnp.float32)], axis=1)


def _pack(f_in, pos):
    n = f_in.shape[0]
    bn = 2000
    return pl.pallas_call(
        _pack_body,
        grid=(n // bn,),
        in_specs=[
            pl.BlockSpec((bn, 162), lambda i: (i, 0)),
            pl.BlockSpec((bn, 3), lambda i: (i, 0)),
            pl.BlockSpec((162, 35), lambda i: (0, 0)),
        ],
        out_specs=[
            pl.BlockSpec((bn, PD), lambda i: (i, 0)),
            pl.BlockSpec((bn, CD), lambda i: (i, 0)),
        ],
        out_shape=[
            jax.ShapeDtypeStruct((n, PD), jnp.float32),
            jax.ShapeDtypeStruct((n, CD), jnp.float32),
        ],
    )(f_in, pos, jnp.asarray(_sel_matrix()))


GCH = 5
GROWS = 195


def _gather_body(p_hbm, pt_hbm, row2, col2, g_hbm,
                 ridx, cidx, rows_v, crows_v, sem, semw):
    w = lax.axis_index("c") * NS + lax.axis_index("s")
    rbase = w * GROWS + jnp.minimum(w, 10)
    extra = w < 10

    @pl.when(extra)
    def _():
        pltpu.sync_copy(row2.at[pl.ds(rbase, GROWS + 1)], ridx)
        pltpu.sync_copy(col2.at[pl.ds(rbase, GROWS + 1)], cidx)

    @pl.when(jnp.logical_not(extra))
    def _():
        pltpu.sync_copy(row2.at[pl.ds(rbase, GROWS)], ridx.at[pl.ds(0, GROWS)])
        pltpu.sync_copy(col2.at[pl.ds(rbase, GROWS)], cidx.at[pl.ds(0, GROWS)])

    def do_rows(r0, k):
        ds_ = []
        for j in range(k):
            ds_.append(pltpu.async_copy(p_hbm.at[ridx.at[r0 + j]],
                                        rows_v.at[pl.ds(j * 128, 128)], sem))
            ds_.append(pltpu.async_copy(pt_hbm.at[cidx.at[r0 + j]],
                                        crows_v.at[pl.ds(j * 128, 128)], sem))
        for d in ds_:
            d.wait()
        e0 = (rbase + r0) * 128
        d1 = pltpu.async_copy(rows_v.at[pl.ds(0, k * 128)],
                              g_hbm.at[pl.ds(e0, k * 128), pl.ds(0, PD)], semw)
        d2 = pltpu.async_copy(crows_v.at[pl.ds(0, k * 128)],
                              g_hbm.at[pl.ds(e0, k * 128), pl.ds(PD, CD)], semw)
        d1.wait()
        d2.wait()

    def body(m, carry):
        do_rows(m * GCH, GCH)
        return carry

    lax.fori_loop(0, GROWS // GCH, body, 0)

    @pl.when(extra)
    def _():
        do_rows(GROWS, 1)


def _gather(P, PT, row2, col2):
    e = row2.shape[0] * row2.shape[1]
    mesh = plsc.VectorSubcoreMesh(core_axis_name="c", subcore_axis_name="s")
    scratch = [
        pltpu.VMEM((GROWS + 1, 128), jnp.int32),
        pltpu.VMEM((GROWS + 1, 128), jnp.int32),
        pltpu.VMEM((GCH * 128, PD), jnp.float32),
        pltpu.VMEM((GCH * 128, CD), jnp.float32),
        pltpu.SemaphoreType.DMA,
        pltpu.SemaphoreType.DMA,
    ]
    kfn = functools.partial(
        pl.kernel,
        mesh=mesh,
        out_type=jax.ShapeDtypeStruct((e, 128), jnp.float32),
        scratch_types=scratch,
        compiler_params=pltpu.CompilerParams(use_tc_tiling_on_sc=False),
    )(_gather_body)
    return kfn(P, PT, row2, col2)


def _dense_body(g_ref, vals_ref, istep_ref, a1_ref, a2s_ref, a2p_ref, a2d_ref,
                s_ref):
    be = g_ref.shape[0]
    gt = g_ref[:, 0:64].T

    v = gt[35:38, :] - gt[PD:PD + 3, :]
    n2 = jnp.sum(v * v, axis=0, keepdims=True) + 1e-12
    ln = jnp.sqrt(n2)
    inv = 1.0 / ln
    x, y, z = v[0:1, :] * inv, v[1:2, :] * inv, v[2:3, :] * inv
    s3 = math.sqrt(3.0)
    s15 = math.sqrt(15.0)
    sh1 = (s3 * x, s3 * y, s3 * z)
    sh2 = (s15 * x * y, s15 * y * z, (math.sqrt(5.0) / 2.0) * (3.0 * z * z - 1.0),
           s15 * x * z, (s15 / 2.0) * (x * x - y * y))

    lnb = jnp.broadcast_to(ln, (10, be))
    diff = (lnb - vals_ref[...]) * istep_ref[...]
    def sus(t):
        safe = jnp.where(t > 0.0, t, 1.0)
        return jnp.where(t > 0.0, jnp.exp(-1.0 / safe), 0.0)
    cemb = 1.14136 * math.exp(2.0) * math.sqrt(10.0)
    emb = cemb * sus(diff + 1.0) * sus(1.0 - diff)

    h = jnp.maximum(jnp.dot(a1_ref[...], emb), 0.0)
    ws = jnp.dot(a2s_ref[...], h[0:64, :])
    wp = jnp.dot(a2p_ref[...], h[64:128, :])
    wd = jnp.dot(a2d_ref[...], h[128:192, :])

    out_s = jnp.broadcast_to(gt[0:1, :], (16, be)) * ws
    out_p = jnp.zeros((16, be), jnp.float32)
    for u in range(3):
        dp = (gt[1 + 3 * u:2 + 3 * u, :] * sh1[0]
              + gt[2 + 3 * u:3 + 3 * u, :] * sh1[1]
              + gt[3 + 3 * u:4 + 3 * u, :] * sh1[2])
        out_p = out_p + jnp.broadcast_to(dp, (16, be)) * wp[16 * u:16 * u + 16, :]
    out_d = jnp.zeros((16, be), jnp.float32)
    for u in range(5):
        dd = jnp.zeros((1, be), jnp.float32)
        for m in range(5):
            dd = dd + gt[10 + 5 * u + m:11 + 5 * u + m, :] * sh2[m]
        out_d = out_d + jnp.broadcast_to(dd, (16, be)) * wd[16 * u:16 * u + 16, :]

    outt = jnp.concatenate([out_s, out_p, out_d], axis=0)
    out = outt.T
    s_ref[...] = jnp.concatenate(
        [out, jnp.zeros((be, 128 - 2 * SD), jnp.float32)], axis=1)


def _dense(G, vals, istep, a1, a2s, a2p, a2d):
    e = G.shape[0]
    be = 1600
    return pl.pallas_call(
        _dense_body,
        grid=(e // be,),
        in_specs=[
            pl.BlockSpec((be, 128), lambda i: (i, 0)),
            pl.BlockSpec((10, 1), lambda i: (0, 0)),
            pl.BlockSpec((1, 1), lambda i: (0, 0)),
            pl.BlockSpec((192, 10), lambda i: (0, 0)),
            pl.BlockSpec((16, 64), lambda i: (0, 0)),
            pl.BlockSpec((48, 64), lambda i: (0, 0)),
            pl.BlockSpec((80, 64), lambda i: (0, 0)),
        ],
        out_specs=[
            pl.BlockSpec((be, 128), lambda i: (i, 0)),
        ],
        out_shape=[
            jax.ShapeDtypeStruct((e, 128), jnp.float32),
        ],
    )(G, vals, istep, a1, a2s, a2p, a2d)[0]


SCH = 5
SROWS = 390
SBROWS = 65
SNBLK = SROWS // SBROWS


def _scatter_body(s_hbm, col2, z_hbm, o_hbm,
                  acc, cidx, rows_v, sem, sema,
                  n_per_s, n):
    c = lax.axis_index("c")
    s = lax.axis_index("s")
    r0 = s * n_per_s
    n_last = n - (NS - 1) * n_per_s

    @pl.when(s < NS - 1)
    def _():
        pltpu.sync_copy(z_hbm.at[pl.ds(0, n_per_s)], acc.at[pl.ds(r0, n_per_s)])

    @pl.when(s == NS - 1)
    def _():
        pltpu.sync_copy(z_hbm.at[pl.ds(0, n_last)], acc.at[pl.ds(r0, n_last)])

    rbase = s * SROWS + jnp.minimum(s, 10)
    extra = s < 10

    plsc.subcore_barrier()
    coff = c * SD

    def do_rows(gr0, lr0, k):
        e0 = (rbase + gr0) * 128
        pltpu.async_copy(s_hbm.at[pl.ds(e0, k * 128), pl.ds(coff, SD)],
                         rows_v.at[pl.ds(0, k * 128)], sem).wait()
        ds_ = []
        for j in range(k):
            ds_.append(pltpu.async_copy(rows_v.at[pl.ds(j * 128, 128)],
                                        acc.at[cidx.at[lr0 + j]], sema, add=True))
        for d in ds_:
            d.wait()

    def blk(b, carry):
        last = jnp.logical_and(b == SNBLK - 1, extra)

        @pl.when(last)
        def _():
            pltpu.sync_copy(col2.at[pl.ds(rbase + b * SBROWS, SBROWS + 1)], cidx)

        @pl.when(jnp.logical_not(last))
        def _():
            pltpu.sync_copy(col2.at[pl.ds(rbase + b * SBROWS, SBROWS)],
                            cidx.at[pl.ds(0, SBROWS)])

        def body(m, carry2):
            do_rows(b * SBROWS + m * SCH, m * SCH, SCH)
            return carry2

        lax.fori_loop(0, SBROWS // SCH, body, 0)

        @pl.when(last)
        def _():
            do_rows(SROWS, SBROWS, 1)

        return carry

    lax.fori_loop(0, SNBLK, blk, 0)

    plsc.subcore_barrier()

    @pl.when(s < NS - 1)
    def _():
        pltpu.sync_copy(acc.at[pl.ds(r0, n_per_s)],
                        o_hbm.at[pl.ds(r0, n_per_s), pl.ds(coff, SD)])

    @pl.when(s == NS - 1)
    def _():
        pltpu.sync_copy(acc.at[pl.ds(r0, n_last)],
                        o_hbm.at[pl.ds(r0, n_last), pl.ds(coff, SD)])


def _scatter(S, col2, n):
    n_per_s = -(-n // NS)
    n_per_s += (-n_per_s) % 8
    assert (NS - 1) * n_per_s < n

    z = jnp.zeros((n_per_s, SD), jnp.float32)
    mesh = plsc.VectorSubcoreMesh(core_axis_name="c", subcore_axis_name="s")
    scratch = [
        pltpu.VMEM_SHARED((n, SD), jnp.float32),
        pltpu.VMEM((SBROWS + 1, 128), jnp.int32),
        pltpu.VMEM((SCH * 128, SD), jnp.float32),
        pltpu.SemaphoreType.DMA,
        pltpu.SemaphoreType.DMA,
    ]
    kfn = functools.partial(
        pl.kernel,
        mesh=mesh,
        out_type=jax.ShapeDtypeStruct((n, 128), jnp.float32),
        scratch_types=scratch,
        compiler_params=pltpu.CompilerParams(use_tc_tiling_on_sc=False),
    )(functools.partial(_scatter_body, n_per_s=n_per_s, n=n))
    return kfn(S, col2, z)


def kernel(f_in, edge_index, pos, max_radius, num_nodes, W1s, W2s, W1p, W2p, W1d, W2d):
    n = f_in.shape[0]
    e = edge_index.shape[1]
    row2 = edge_index[0].reshape(-1, 128)
    col2 = edge_index[1].reshape(-1, 128)

    num_neighbors = e / n
    a1 = (jnp.concatenate([W1s, W1p, W1d], axis=1) * (1.0 / math.sqrt(10.0))).T
    cs = math.sqrt(2.0) / 8.0 / math.sqrt(num_neighbors)
    a2s = (W2s * cs).T
    a2p = (W2p * (cs / 3.0)).T
    a2d = (W2d * (cs / 5.0)).T

    mr = jnp.asarray(max_radius, jnp.float32)
    step = mr / (10 + 1)
    vals = (jnp.arange(1, 11, dtype=jnp.float32) * step).reshape(10, 1)
    istep = (1.0 / step).reshape(1, 1)

    P, PT = _pack(f_in, pos)
    G = _gather(P, PT, row2, col2)
    S = _dense(G, vals, istep, a1, a2s, a2p, a2d)
    O = _scatter(S, col2, n)
    return O[:, 0:2 * SD]


# speedup vs baseline: 21.9522x; 1.1551x over previous
import functools
import math

import jax
import jax.numpy as jnp
import numpy as np
from jax import lax
from jax.experimental import pallas as pl
from jax.experimental.pallas import tpu as pltpu
from jax.experimental.pallas import tpu_sc as plsc

NC = 2
NS = 16
CH = 128

PD = 48
CD = 16
SD = 24


def _sel_matrix():
    srcs = ([0]
            + [(1 + u) * 9 + (1 + m) for u in range(3) for m in range(3)]
            + [(4 + u) * 9 + (4 + m) for u in range(5) for m in range(5)])
    sel = np.zeros((162, 35), np.float32)
    for j, k in enumerate(srcs):
        sel[2 * k, j] = 1.0
        sel[2 * k + 1, j] = 1.0
    return sel


def _pack_body(f_ref, pos_ref, sel_ref, p_ref, pt_ref):
    f = f_ref[...]
    bn = f.shape[0]
    pcols = jnp.dot(f, sel_ref[...], precision=lax.Precision.HIGHEST)
    posb = pos_ref[...]
    zp = jnp.zeros((bn, PD - 38), jnp.float32)
    p_ref[...] = jnp.concatenate([pcols, posb, zp], axis=1)
    pt_ref[...] = jnp.concatenate([posb, jnp.zeros((bn, CD - 3), jnp.float32)], axis=1)


def _pack(f_in, pos):
    n = f_in.shape[0]
    bn = 2000
    return pl.pallas_call(
        _pack_body,
        grid=(n // bn,),
        in_specs=[
            pl.BlockSpec((bn, 162), lambda i: (i, 0)),
            pl.BlockSpec((bn, 3), lambda i: (i, 0)),
            pl.BlockSpec((162, 35), lambda i: (0, 0)),
        ],
        out_specs=[
            pl.BlockSpec((bn, PD), lambda i: (i, 0)),
            pl.BlockSpec((bn, CD), lambda i: (i, 0)),
        ],
        out_shape=[
            jax.ShapeDtypeStruct((n, PD), jnp.float32),
            jax.ShapeDtypeStruct((n, CD), jnp.float32),
        ],
    )(f_in, pos, jnp.asarray(_sel_matrix()))


GCH = 5
GROWS = 195


def _gather_body(p_hbm, pt_hbm, row2, col2, g_hbm,
                 ridx, cidx, rows_v, crows_v, sem, semw):
    w = lax.axis_index("c") * NS + lax.axis_index("s")
    rbase = w * GROWS + jnp.minimum(w, 10)
    extra = w < 10

    @pl.when(extra)
    def _():
        pltpu.sync_copy(row2.at[pl.ds(rbase, GROWS + 1)], ridx)
        pltpu.sync_copy(col2.at[pl.ds(rbase, GROWS + 1)], cidx)

    @pl.when(jnp.logical_not(extra))
    def _():
        pltpu.sync_copy(row2.at[pl.ds(rbase, GROWS)], ridx.at[pl.ds(0, GROWS)])
        pltpu.sync_copy(col2.at[pl.ds(rbase, GROWS)], cidx.at[pl.ds(0, GROWS)])

    def do_rows(r0, k):
        ds_ = []
        for j in range(k):
            ds_.append(pltpu.async_copy(p_hbm.at[ridx.at[r0 + j]],
                                        rows_v.at[pl.ds(j * 128, 128)], sem))
            ds_.append(pltpu.async_copy(pt_hbm.at[cidx.at[r0 + j]],
                                        crows_v.at[pl.ds(j * 128, 128)], sem))
        for d in ds_:
            d.wait()
        e0 = (rbase + r0) * 128
        d1 = pltpu.async_copy(rows_v.at[pl.ds(0, k * 128)],
                              g_hbm.at[pl.ds(e0, k * 128), pl.ds(0, PD)], semw)
        d2 = pltpu.async_copy(crows_v.at[pl.ds(0, k * 128)],
                              g_hbm.at[pl.ds(e0, k * 128), pl.ds(PD, CD)], semw)
        d1.wait()
        d2.wait()

    def body(m, carry):
        do_rows(m * GCH, GCH)
        return carry

    lax.fori_loop(0, GROWS // GCH, body, 0)

    @pl.when(extra)
    def _():
        do_rows(GROWS, 1)


def _gather(P, PT, row2, col2):
    e = row2.shape[0] * row2.shape[1]
    mesh = plsc.VectorSubcoreMesh(core_axis_name="c", subcore_axis_name="s")
    scratch = [
        pltpu.VMEM((GROWS + 1, 128), jnp.int32),
        pltpu.VMEM((GROWS + 1, 128), jnp.int32),
        pltpu.VMEM((GCH * 128, PD), jnp.float32),
        pltpu.VMEM((GCH * 128, CD), jnp.float32),
        pltpu.SemaphoreType.DMA,
        pltpu.SemaphoreType.DMA,
    ]
    kfn = functools.partial(
        pl.kernel,
        mesh=mesh,
        out_type=jax.ShapeDtypeStruct((e, 128), jnp.float32),
        scratch_types=scratch,
        compiler_params=pltpu.CompilerParams(use_tc_tiling_on_sc=False),
    )(_gather_body)
    return kfn(P, PT, row2, col2)


def _exp_red_mats():
    expm = np.zeros((34, 8), np.float32)
    red = np.zeros((8, 34), np.float32)
    for r in range(9):
        expm[r, r % 3] = 1.0
        red[r // 3, r] = 1.0
    for r in range(9, 34):
        expm[r, 3 + (r - 9) % 5] = 1.0
        red[3 + (r - 9) // 5, r] = 1.0
    return expm, red


def _dense_body(g_ref, vals_ref, istep_ref, a1_ref, a2s_ref, a2p_ref, a2d_ref,
                expm_ref, red_ref, s_ref):
    be = g_ref.shape[0]
    gt = g_ref[:, 0:64].T

    v = gt[35:38, :] - gt[PD:PD + 3, :]
    n2 = jnp.sum(v * v, axis=0, keepdims=True) + 1e-12
    ln = jnp.sqrt(n2)
    inv = 1.0 / ln
    x, y, z = v[0:1, :] * inv, v[1:2, :] * inv, v[2:3, :] * inv
    s3 = math.sqrt(3.0)
    s15 = math.sqrt(15.0)
    shvec = jnp.concatenate([
        s3 * x, s3 * y, s3 * z,
        s15 * x * y, s15 * y * z, (math.sqrt(5.0) / 2.0) * (3.0 * z * z - 1.0),
        s15 * x * z, (s15 / 2.0) * (x * x - y * y)], axis=0)

    lnb = jnp.broadcast_to(ln, (10, be))
    diff = (lnb - vals_ref[...]) * istep_ref[...]
    def sus(t):
        safe = jnp.where(t > 0.0, t, 1.0)
        return jnp.where(t > 0.0, jnp.exp(-1.0 / safe), 0.0)
    cemb = 1.14136 * math.exp(2.0) * math.sqrt(10.0)
    emb = cemb * sus(diff + 1.0) * sus(1.0 - diff)

    h = jnp.maximum(jnp.dot(a1_ref[...], emb), 0.0)
    ws = jnp.dot(a2s_ref[...], h[0:64, :])
    wp = jnp.dot(a2p_ref[...], h[64:128, :])
    wd = jnp.dot(a2d_ref[...], h[128:192, :])

    shb = jnp.dot(expm_ref[...], shvec)
    prod = gt[1:35, :] * shb
    dpd = jnp.dot(red_ref[...], prod)

    out_s = jnp.broadcast_to(gt[0:1, :], (16, be)) * ws
    out_p = jnp.zeros((16, be), jnp.float32)
    for u in range(3):
        out_p = out_p + (jnp.broadcast_to(dpd[u:u + 1, :], (16, be))
                         * wp[16 * u:16 * u + 16, :])
    out_d = jnp.zeros((16, be), jnp.float32)
    for u in range(5):
        out_d = out_d + (jnp.broadcast_to(dpd[3 + u:4 + u, :], (16, be))
                         * wd[16 * u:16 * u + 16, :])

    outt = jnp.concatenate([out_s, out_p, out_d], axis=0)
    out = outt.T
    s_ref[...] = jnp.concatenate(
        [out, jnp.zeros((be, 128 - 2 * SD), jnp.float32)], axis=1)


def _dense(G, vals, istep, a1, a2s, a2p, a2d):
    e = G.shape[0]
    be = 3200
    expm, red = _exp_red_mats()
    return pl.pallas_call(
        _dense_body,
        grid=(e // be,),
        in_specs=[
            pl.BlockSpec((be, 128), lambda i: (i, 0)),
            pl.BlockSpec((10, 1), lambda i: (0, 0)),
            pl.BlockSpec((1, 1), lambda i: (0, 0)),
            pl.BlockSpec((192, 10), lambda i: (0, 0)),
            pl.BlockSpec((16, 64), lambda i: (0, 0)),
            pl.BlockSpec((48, 64), lambda i: (0, 0)),
            pl.BlockSpec((80, 64), lambda i: (0, 0)),
            pl.BlockSpec((34, 8), lambda i: (0, 0)),
            pl.BlockSpec((8, 34), lambda i: (0, 0)),
        ],
        out_specs=[
            pl.BlockSpec((be, 128), lambda i: (i, 0)),
        ],
        out_shape=[
            jax.ShapeDtypeStruct((e, 128), jnp.float32),
        ],
    )(G, vals, istep, a1, a2s, a2p, a2d,
      jnp.asarray(expm), jnp.asarray(red))[0]


SCH = 5
SROWS = 390
SBROWS = 65
SNBLK = SROWS // SBROWS


def _scatter_body(s_hbm, col2, z_hbm, o_hbm,
                  acc, cidx, rows_v, sem, sema,
                  n_per_s, n):
    c = lax.axis_index("c")
    s = lax.axis_index("s")
    r0 = s * n_per_s
    n_last = n - (NS - 1) * n_per_s

    @pl.when(s < NS - 1)
    def _():
        pltpu.sync_copy(z_hbm.at[pl.ds(0, n_per_s)], acc.at[pl.ds(r0, n_per_s)])

    @pl.when(s == NS - 1)
    def _():
        pltpu.sync_copy(z_hbm.at[pl.ds(0, n_last)], acc.at[pl.ds(r0, n_last)])

    rbase = s * SROWS + jnp.minimum(s, 10)
    extra = s < 10

    plsc.subcore_barrier()
    coff = c * SD

    def do_rows(gr0, lr0, k):
        e0 = (rbase + gr0) * 128
        pltpu.async_copy(s_hbm.at[pl.ds(e0, k * 128), pl.ds(coff, SD)],
                         rows_v.at[pl.ds(0, k * 128)], sem).wait()
        ds_ = []
        for j in range(k):
            ds_.append(pltpu.async_copy(rows_v.at[pl.ds(j * 128, 128)],
                                        acc.at[cidx.at[lr0 + j]], sema, add=True))
        for d in ds_:
            d.wait()

    def blk(b, carry):
        last = jnp.logical_and(b == SNBLK - 1, extra)

        @pl.when(last)
        def _():
            pltpu.sync_copy(col2.at[pl.ds(rbase + b * SBROWS, SBROWS + 1)], cidx)

        @pl.when(jnp.logical_not(last))
        def _():
            pltpu.sync_copy(col2.at[pl.ds(rbase + b * SBROWS, SBROWS)],
                            cidx.at[pl.ds(0, SBROWS)])

        def body(m, carry2):
            do_rows(b * SBROWS + m * SCH, m * SCH, SCH)
            return carry2

        lax.fori_loop(0, SBROWS // SCH, body, 0)

        @pl.when(last)
        def _():
            do_rows(SROWS, SBROWS, 1)

        return carry

    lax.fori_loop(0, SNBLK, blk, 0)

    plsc.subcore_barrier()

    @pl.when(s < NS - 1)
    def _():
        pltpu.sync_copy(acc.at[pl.ds(r0, n_per_s)],
                        o_hbm.at[pl.ds(r0, n_per_s), pl.ds(coff, SD)])

    @pl.when(s == NS - 1)
    def _():
        pltpu.sync_copy(acc.at[pl.ds(r0, n_last)],
                        o_hbm.at[pl.ds(r0, n_last), pl.ds(coff, SD)])


def _scatter(S, col2, n):
    n_per_s = -(-n // NS)
    n_per_s += (-n_per_s) % 8
    assert (NS - 1) * n_per_s < n

    z = jnp.zeros((n_per_s, SD), jnp.float32)
    mesh = plsc.VectorSubcoreMesh(core_axis_name="c", subcore_axis_name="s")
    scratch = [
        pltpu.VMEM_SHARED((n, SD), jnp.float32),
        pltpu.VMEM((SBROWS + 1, 128), jnp.int32),
        pltpu.VMEM((SCH * 128, SD), jnp.float32),
        pltpu.SemaphoreType.DMA,
        pltpu.SemaphoreType.DMA,
    ]
    kfn = functools.partial(
        pl.kernel,
        mesh=mesh,
        out_type=jax.ShapeDtypeStruct((n, 128), jnp.float32),
        scratch_types=scratch,
        compiler_params=pltpu.CompilerParams(use_tc_tiling_on_sc=False),
    )(functools.partial(_scatter_body, n_per_s=n_per_s, n=n))
    return kfn(S, col2, z)


def kernel(f_in, edge_index, pos, max_radius, num_nodes, W1s, W2s, W1p, W2p, W1d, W2d):
    n = f_in.shape[0]
    e = edge_index.shape[1]
    row2 = edge_index[0].reshape(-1, 128)
    col2 = edge_index[1].reshape(-1, 128)

    num_neighbors = e / n
    a1 = (jnp.concatenate([W1s, W1p, W1d], axis=1) * (1.0 / math.sqrt(10.0))).T
    cs = math.sqrt(2.0) / 8.0 / math.sqrt(num_neighbors)
    a2s = (W2s * cs).T
    a2p = (W2p * (cs / 3.0)).T
    a2d = (W2d * (cs / 5.0)).T

    mr = jnp.asarray(max_radius, jnp.float32)
    step = mr / (10 + 1)
    vals = (jnp.arange(1, 11, dtype=jnp.float32) * step).reshape(10, 1)
    istep = (1.0 / step).reshape(1, 1)

    P, PT = _pack(f_in, pos)
    G = _gather(P, PT, row2, col2)
    S = _dense(G, vals, istep, a1, a2s, a2p, a2d)
    O = _scatter(S, col2, n)
    return O[:, 0:2 * SD]


# speedup vs baseline: 24.4659x; 1.1145x over previous
import functools
import math

import jax
import jax.numpy as jnp
import numpy as np
from jax import lax
from jax.experimental import pallas as pl
from jax.experimental.pallas import tpu as pltpu
from jax.experimental.pallas import tpu_sc as plsc

NC = 2
NS = 16
CH = 128

PD = 48
CD = 16
SD = 24


def _sel_matrix():
    srcs = ([0]
            + [(1 + u) * 9 + (1 + m) for u in range(3) for m in range(3)]
            + [(4 + u) * 9 + (4 + m) for u in range(5) for m in range(5)])
    sel = np.zeros((162, 35), np.float32)
    for j, k in enumerate(srcs):
        sel[2 * k, j] = 1.0
        sel[2 * k + 1, j] = 1.0
    return sel


def _pack_body(f_ref, pos_ref, sel_ref, p_ref, pt_ref):
    f = f_ref[...]
    bn = f.shape[0]
    pcols = jnp.dot(f, sel_ref[...], precision=lax.Precision.HIGHEST)
    posb = pos_ref[...]
    zp = jnp.zeros((bn, PD - 38), jnp.float32)
    p_ref[...] = jnp.concatenate([pcols, posb, zp], axis=1)
    pt_ref[...] = jnp.concatenate([posb, jnp.zeros((bn, CD - 3), jnp.float32)], axis=1)


def _pack(f_in, pos):
    n = f_in.shape[0]
    bn = 2000
    return pl.pallas_call(
        _pack_body,
        grid=(n // bn,),
        in_specs=[
            pl.BlockSpec((bn, 162), lambda i: (i, 0)),
            pl.BlockSpec((bn, 3), lambda i: (i, 0)),
            pl.BlockSpec((162, 35), lambda i: (0, 0)),
        ],
        out_specs=[
            pl.BlockSpec((bn, PD), lambda i: (i, 0)),
            pl.BlockSpec((bn, CD), lambda i: (i, 0)),
        ],
        out_shape=[
            jax.ShapeDtypeStruct((n, PD), jnp.float32),
            jax.ShapeDtypeStruct((n, CD), jnp.float32),
        ],
    )(f_in, pos, jnp.asarray(_sel_matrix()))


GCH = 5


def _gather_body(p_hbm, pt_hbm, row2, col2, g_hbm,
                 ridx, cidx, rows_v, crows_v, sem, semw, grows, nx):
    w = lax.axis_index("c") * NS + lax.axis_index("s")
    rbase = w * grows + jnp.minimum(w, nx)
    extra = w < nx

    @pl.when(extra)
    def _():
        pltpu.sync_copy(row2.at[pl.ds(rbase, grows + 1)], ridx)
        pltpu.sync_copy(col2.at[pl.ds(rbase, grows + 1)], cidx)

    @pl.when(jnp.logical_not(extra))
    def _():
        pltpu.sync_copy(row2.at[pl.ds(rbase, grows)], ridx.at[pl.ds(0, grows)])
        pltpu.sync_copy(col2.at[pl.ds(rbase, grows)], cidx.at[pl.ds(0, grows)])

    def do_rows(r0, k):
        ds_ = []
        for j in range(k):
            ds_.append(pltpu.async_copy(p_hbm.at[ridx.at[r0 + j]],
                                        rows_v.at[pl.ds(j * 128, 128)], sem))
            ds_.append(pltpu.async_copy(pt_hbm.at[cidx.at[r0 + j]],
                                        crows_v.at[pl.ds(j * 128, 128)], sem))
        for d in ds_:
            d.wait()
        e0 = (rbase + r0) * 128
        d1 = pltpu.async_copy(rows_v.at[pl.ds(0, k * 128)],
                              g_hbm.at[pl.ds(e0, k * 128), pl.ds(0, PD)], semw)
        d2 = pltpu.async_copy(crows_v.at[pl.ds(0, k * 128)],
                              g_hbm.at[pl.ds(e0, k * 128), pl.ds(PD, CD)], semw)
        d1.wait()
        d2.wait()

    def body(m, carry):
        do_rows(m * GCH, GCH)
        return carry

    lax.fori_loop(0, grows // GCH, body, 0)
    for r in range(grows % GCH):
        do_rows((grows // GCH) * GCH + r, 1)

    @pl.when(extra)
    def _():
        do_rows(grows, 1)


def _gather(P, PT, row2, col2):
    rows = row2.shape[0]
    e = rows * row2.shape[1]
    nw = NC * NS
    grows, nx = rows // nw, rows % nw
    mesh = plsc.VectorSubcoreMesh(core_axis_name="c", subcore_axis_name="s")
    scratch = [
        pltpu.VMEM((grows + 1, 128), jnp.int32),
        pltpu.VMEM((grows + 1, 128), jnp.int32),
        pltpu.VMEM((GCH * 128, PD), jnp.float32),
        pltpu.VMEM((GCH * 128, CD), jnp.float32),
        pltpu.SemaphoreType.DMA,
        pltpu.SemaphoreType.DMA,
    ]
    kfn = functools.partial(
        pl.kernel,
        mesh=mesh,
        out_type=jax.ShapeDtypeStruct((e, 128), jnp.float32),
        scratch_types=scratch,
        compiler_params=pltpu.CompilerParams(use_tc_tiling_on_sc=False),
    )(functools.partial(_gather_body, grows=grows, nx=nx))
    return kfn(P, PT, row2, col2)


def _exp_red_mats():
    expm = np.zeros((34, 8), np.float32)
    red = np.zeros((8, 34), np.float32)
    for r in range(9):
        expm[r, r % 3] = 1.0
        red[r // 3, r] = 1.0
    for r in range(9, 34):
        expm[r, 3 + (r - 9) % 5] = 1.0
        red[3 + (r - 9) // 5, r] = 1.0
    return expm, red


def _dense_body(g_ref, vals_ref, istep_ref, a1_ref, a2s_ref, a2p_ref, a2d_ref,
                expm_ref, red_ref, s_ref):
    be = g_ref.shape[0]
    gt = g_ref[:, 0:64].T

    v = gt[35:38, :] - gt[PD:PD + 3, :]
    n2 = jnp.sum(v * v, axis=0, keepdims=True) + 1e-12
    ln = jnp.sqrt(n2)
    inv = 1.0 / ln
    x, y, z = v[0:1, :] * inv, v[1:2, :] * inv, v[2:3, :] * inv
    s3 = math.sqrt(3.0)
    s15 = math.sqrt(15.0)
    shvec = jnp.concatenate([
        s3 * x, s3 * y, s3 * z,
        s15 * x * y, s15 * y * z, (math.sqrt(5.0) / 2.0) * (3.0 * z * z - 1.0),
        s15 * x * z, (s15 / 2.0) * (x * x - y * y)], axis=0)

    lnb = jnp.broadcast_to(ln, (10, be))
    diff = (lnb - vals_ref[...]) * istep_ref[...]
    def sus(t):
        safe = jnp.where(t > 0.0, t, 1.0)
        return jnp.where(t > 0.0, jnp.exp(-1.0 / safe), 0.0)
    cemb = 1.14136 * math.exp(2.0) * math.sqrt(10.0)
    emb = cemb * sus(diff + 1.0) * sus(1.0 - diff)

    h = jnp.maximum(jnp.dot(a1_ref[...], emb), 0.0)
    ws = jnp.dot(a2s_ref[...], h[0:64, :])
    wp = jnp.dot(a2p_ref[...], h[64:128, :])
    wd = jnp.dot(a2d_ref[...], h[128:192, :])

    shb = jnp.dot(expm_ref[...], shvec)
    prod = gt[1:35, :] * shb
    dpd = jnp.dot(red_ref[...], prod)

    out_s = jnp.broadcast_to(gt[0:1, :], (16, be)) * ws
    out_p = jnp.zeros((16, be), jnp.float32)
    for u in range(3):
        out_p = out_p + (jnp.broadcast_to(dpd[u:u + 1, :], (16, be))
                         * wp[16 * u:16 * u + 16, :])
    out_d = jnp.zeros((16, be), jnp.float32)
    for u in range(5):
        out_d = out_d + (jnp.broadcast_to(dpd[3 + u:4 + u, :], (16, be))
                         * wd[16 * u:16 * u + 16, :])

    outt = jnp.concatenate([out_s, out_p, out_d], axis=0)
    out = outt.T
    s_ref[...] = jnp.concatenate(
        [out, jnp.zeros((be, 128 - 2 * SD), jnp.float32)], axis=1)


def _dense(G, vals, istep, a1, a2s, a2p, a2d):
    e = G.shape[0]
    be = 3200
    expm, red = _exp_red_mats()
    return pl.pallas_call(
        _dense_body,
        grid=(e // be,),
        in_specs=[
            pl.BlockSpec((be, 128), lambda i: (i, 0)),
            pl.BlockSpec((10, 1), lambda i: (0, 0)),
            pl.BlockSpec((1, 1), lambda i: (0, 0)),
            pl.BlockSpec((192, 10), lambda i: (0, 0)),
            pl.BlockSpec((16, 64), lambda i: (0, 0)),
            pl.BlockSpec((48, 64), lambda i: (0, 0)),
            pl.BlockSpec((80, 64), lambda i: (0, 0)),
            pl.BlockSpec((34, 8), lambda i: (0, 0)),
            pl.BlockSpec((8, 34), lambda i: (0, 0)),
        ],
        out_specs=[
            pl.BlockSpec((be, 128), lambda i: (i, 0)),
        ],
        out_shape=[
            jax.ShapeDtypeStruct((e, 128), jnp.float32),
        ],
    )(G, vals, istep, a1, a2s, a2p, a2d,
      jnp.asarray(expm), jnp.asarray(red))[0]


SCH = 5
SBROWS = 65


def _scatter_body(s_hbm, col2, z_hbm, o_hbm,
                  acc, cidx, rows_v, sem, sema,
                  n_per_s, n, srows, snx):
    c = lax.axis_index("c")
    s = lax.axis_index("s")
    r0 = s * n_per_s
    n_last = n - (NS - 1) * n_per_s

    @pl.when(s < NS - 1)
    def _():
        pltpu.sync_copy(z_hbm.at[pl.ds(0, n_per_s)], acc.at[pl.ds(r0, n_per_s)])

    @pl.when(s == NS - 1)
    def _():
        pltpu.sync_copy(z_hbm.at[pl.ds(0, n_last)], acc.at[pl.ds(r0, n_last)])

    rbase = s * srows + jnp.minimum(s, snx)
    extra = s < snx
    snblk = srows // SBROWS

    plsc.subcore_barrier()
    coff = c * SD

    def do_rows(gr0, lr0, k):
        e0 = (rbase + gr0) * 128
        pltpu.async_copy(s_hbm.at[pl.ds(e0, k * 128), pl.ds(coff, SD)],
                         rows_v.at[pl.ds(0, k * 128)], sem).wait()
        ds_ = []
        for j in range(k):
            ds_.append(pltpu.async_copy(rows_v.at[pl.ds(j * 128, 128)],
                                        acc.at[cidx.at[lr0 + j]], sema, add=True))
        for d in ds_:
            d.wait()

    def blk(b, carry):
        last = jnp.logical_and(b == snblk - 1, extra)

        @pl.when(last)
        def _():
            pltpu.sync_copy(col2.at[pl.ds(rbase + b * SBROWS, SBROWS + 1)], cidx)

        @pl.when(jnp.logical_not(last))
        def _():
            pltpu.sync_copy(col2.at[pl.ds(rbase + b * SBROWS, SBROWS)],
                            cidx.at[pl.ds(0, SBROWS)])

        def body(m, carry2):
            do_rows(b * SBROWS + m * SCH, m * SCH, SCH)
            return carry2

        lax.fori_loop(0, SBROWS // SCH, body, 0)

        @pl.when(last)
        def _():
            do_rows(srows, SBROWS, 1)

        return carry

    lax.fori_loop(0, snblk, blk, 0)

    plsc.subcore_barrier()

    @pl.when(s < NS - 1)
    def _():
        pltpu.sync_copy(acc.at[pl.ds(r0, n_per_s)],
                        o_hbm.at[pl.ds(r0, n_per_s), pl.ds(coff, SD)])

    @pl.when(s == NS - 1)
    def _():
        pltpu.sync_copy(acc.at[pl.ds(r0, n_last)],
                        o_hbm.at[pl.ds(r0, n_last), pl.ds(coff, SD)])


def _scatter(S, col2, n):
    rows = col2.shape[0]
    srows, snx = rows // NS, rows % NS
    assert srows % SBROWS == 0
    n_per_s = -(-n // NS)
    n_per_s += (-n_per_s) % 8
    assert (NS - 1) * n_per_s < n

    z = jnp.zeros((n_per_s, SD), jnp.float32)
    mesh = plsc.VectorSubcoreMesh(core_axis_name="c", subcore_axis_name="s")
    scratch = [
        pltpu.VMEM_SHARED((n, SD), jnp.float32),
        pltpu.VMEM((SBROWS + 1, 128), jnp.int32),
        pltpu.VMEM((SCH * 128, SD), jnp.float32),
        pltpu.SemaphoreType.DMA,
        pltpu.SemaphoreType.DMA,
    ]
    kfn = functools.partial(
        pl.kernel,
        mesh=mesh,
        out_type=jax.ShapeDtypeStruct((n, 128), jnp.float32),
        scratch_types=scratch,
        compiler_params=pltpu.CompilerParams(use_tc_tiling_on_sc=False),
    )(functools.partial(_scatter_body, n_per_s=n_per_s, n=n,
                        srows=srows, snx=snx))
    return kfn(S, col2, z)


def kernel(f_in, edge_index, pos, max_radius, num_nodes, W1s, W2s, W1p, W2p, W1d, W2d):
    n = f_in.shape[0]
    e = edge_index.shape[1]
    row2 = edge_index[0].reshape(-1, 128)
    col2 = edge_index[1].reshape(-1, 128)

    num_neighbors = e / n
    a1 = (jnp.concatenate([W1s, W1p, W1d], axis=1) * (1.0 / math.sqrt(10.0))).T
    cs = math.sqrt(2.0) / 8.0 / math.sqrt(num_neighbors)
    a2s = (W2s * cs).T
    a2p = (W2p * (cs / 3.0)).T
    a2d = (W2d * (cs / 5.0)).T

    mr = jnp.asarray(max_radius, jnp.float32)
    step = mr / (10 + 1)
    vals = (jnp.arange(1, 11, dtype=jnp.float32) * step).reshape(10, 1)
    istep = (1.0 / step).reshape(1, 1)

    P, PT = _pack(f_in, pos)
    rh = row2.shape[0] // 2
    O = None
    for h in range(2):
        r2h = row2[h * rh:(h + 1) * rh]
        c2h = col2[h * rh:(h + 1) * rh]
        Gh = _gather(P, PT, r2h, c2h)
        Sh = _dense(Gh, vals, istep, a1, a2s, a2p, a2d)
        Oh = _scatter(Sh, c2h, n)
        O = Oh if O is None else O + Oh
    return O[:, 0:2 * SD]


# speedup vs baseline: 25.8529x; 1.0567x over previous
import functools
import math

import jax
import jax.numpy as jnp
import numpy as np
from jax import lax
from jax.experimental import pallas as pl
from jax.experimental.pallas import tpu as pltpu
from jax.experimental.pallas import tpu_sc as plsc

NC = 2
NS = 16
CH = 128

PD = 48
CD = 16
SD = 24


def _sel_matrix():
    srcs = ([0]
            + [(1 + u) * 9 + (1 + m) for u in range(3) for m in range(3)]
            + [(4 + u) * 9 + (4 + m) for u in range(5) for m in range(5)])
    sel = np.zeros((162, 35), np.float32)
    for j, k in enumerate(srcs):
        sel[2 * k, j] = 1.0
        sel[2 * k + 1, j] = 1.0
    return sel


def _pack_body(f_ref, pos_ref, sel_ref, p_ref, pt_ref):
    f = f_ref[...]
    bn = f.shape[0]
    pcols = jnp.dot(f, sel_ref[...], precision=lax.Precision.HIGHEST)
    posb = pos_ref[...]
    zp = jnp.zeros((bn, PD - 38), jnp.float32)
    p_ref[...] = jnp.concatenate([pcols, posb, zp], axis=1)
    pt_ref[...] = jnp.concatenate([posb, jnp.zeros((bn, CD - 3), jnp.float32)], axis=1)


def _pack(f_in, pos):
    n = f_in.shape[0]
    bn = 2000
    return pl.pallas_call(
        _pack_body,
        grid=(n // bn,),
        in_specs=[
            pl.BlockSpec((bn, 162), lambda i: (i, 0)),
            pl.BlockSpec((bn, 3), lambda i: (i, 0)),
            pl.BlockSpec((162, 35), lambda i: (0, 0)),
        ],
        out_specs=[
            pl.BlockSpec((bn, PD), lambda i: (i, 0)),
            pl.BlockSpec((bn, CD), lambda i: (i, 0)),
        ],
        out_shape=[
            jax.ShapeDtypeStruct((n, PD), jnp.float32),
            jax.ShapeDtypeStruct((n, CD), jnp.float32),
        ],
    )(f_in, pos, jnp.asarray(_sel_matrix()))


GCH = 5


def _gather_body(p_hbm, pt_hbm, row2, col2, g_hbm,
                 ridx, cidx, rows_v, crows_v, sem, semw, grows, nx, rofs):
    w = lax.axis_index("c") * NS + lax.axis_index("s")
    rbase = w * grows + jnp.minimum(w, nx)
    hbase = rofs + rbase
    extra = w < nx

    @pl.when(extra)
    def _():
        pltpu.sync_copy(row2.at[pl.ds(hbase, grows + 1)], ridx)
        pltpu.sync_copy(col2.at[pl.ds(hbase, grows + 1)], cidx)

    @pl.when(jnp.logical_not(extra))
    def _():
        pltpu.sync_copy(row2.at[pl.ds(hbase, grows)], ridx.at[pl.ds(0, grows)])
        pltpu.sync_copy(col2.at[pl.ds(hbase, grows)], cidx.at[pl.ds(0, grows)])

    def do_rows(r0, k):
        ds_ = []
        for j in range(k):
            ds_.append(pltpu.async_copy(p_hbm.at[ridx.at[r0 + j]],
                                        rows_v.at[pl.ds(j * 128, 128)], sem))
            ds_.append(pltpu.async_copy(pt_hbm.at[cidx.at[r0 + j]],
                                        crows_v.at[pl.ds(j * 128, 128)], sem))
        for d in ds_:
            d.wait()
        e0 = (rbase + r0) * 128
        d1 = pltpu.async_copy(rows_v.at[pl.ds(0, k * 128)],
                              g_hbm.at[pl.ds(e0, k * 128), pl.ds(0, PD)], semw)
        d2 = pltpu.async_copy(crows_v.at[pl.ds(0, k * 128)],
                              g_hbm.at[pl.ds(e0, k * 128), pl.ds(PD, CD)], semw)
        d1.wait()
        d2.wait()

    def body(m, carry):
        do_rows(m * GCH, GCH)
        return carry

    lax.fori_loop(0, grows // GCH, body, 0)
    for r in range(grows % GCH):
        do_rows((grows // GCH) * GCH + r, 1)

    @pl.when(extra)
    def _():
        do_rows(grows, 1)


def _gather(P, PT, row2, col2, rofs, rows):
    e = rows * row2.shape[1]
    nw = NC * NS
    grows, nx = rows // nw, rows % nw
    assert grows % GCH == 0
    mesh = plsc.VectorSubcoreMesh(core_axis_name="c", subcore_axis_name="s")
    scratch = [
        pltpu.VMEM((grows + 1, 128), jnp.int32),
        pltpu.VMEM((grows + 1, 128), jnp.int32),
        pltpu.VMEM((GCH * 128, PD), jnp.float32),
        pltpu.VMEM((GCH * 128, CD), jnp.float32),
        pltpu.SemaphoreType.DMA,
        pltpu.SemaphoreType.DMA,
    ]
    kfn = functools.partial(
        pl.kernel,
        mesh=mesh,
        out_type=jax.ShapeDtypeStruct((e, 128), jnp.float32),
        scratch_types=scratch,
        compiler_params=pltpu.CompilerParams(use_tc_tiling_on_sc=False),
    )(functools.partial(_gather_body, grows=grows, nx=nx, rofs=rofs))
    return kfn(P, PT, row2, col2)


def _exp_red_mats():
    expm = np.zeros((34, 8), np.float32)
    red = np.zeros((8, 34), np.float32)
    for r in range(9):
        expm[r, r % 3] = 1.0
        red[r // 3, r] = 1.0
    for r in range(9, 34):
        expm[r, 3 + (r - 9) % 5] = 1.0
        red[3 + (r - 9) // 5, r] = 1.0
    return expm, red


def _dense_body(g_ref, vals_ref, istep_ref, a1_ref, a2s_ref, a2p_ref, a2d_ref,
                expm_ref, red_ref, s_ref):
    be = g_ref.shape[0]
    gt = g_ref[:, 0:64].T

    v = gt[35:38, :] - gt[PD:PD + 3, :]
    n2 = jnp.sum(v * v, axis=0, keepdims=True) + 1e-12
    ln = jnp.sqrt(n2)
    inv = 1.0 / ln
    x, y, z = v[0:1, :] * inv, v[1:2, :] * inv, v[2:3, :] * inv
    s3 = math.sqrt(3.0)
    s15 = math.sqrt(15.0)
    shvec = jnp.concatenate([
        s3 * x, s3 * y, s3 * z,
        s15 * x * y, s15 * y * z, (math.sqrt(5.0) / 2.0) * (3.0 * z * z - 1.0),
        s15 * x * z, (s15 / 2.0) * (x * x - y * y)], axis=0)

    lnb = jnp.broadcast_to(ln, (10, be))
    diff = (lnb - vals_ref[...]) * istep_ref[...]
    def sus(t):
        safe = jnp.where(t > 0.0, t, 1.0)
        return jnp.where(t > 0.0, jnp.exp(-1.0 / safe), 0.0)
    cemb = 1.14136 * math.exp(2.0) * math.sqrt(10.0)
    emb = cemb * sus(diff + 1.0) * sus(1.0 - diff)

    h = jnp.maximum(jnp.dot(a1_ref[...], emb), 0.0)
    ws = jnp.dot(a2s_ref[...], h[0:64, :])
    wp = jnp.dot(a2p_ref[...], h[64:128, :])
    wd = jnp.dot(a2d_ref[...], h[128:192, :])

    shb = jnp.dot(expm_ref[...], shvec)
    prod = gt[1:35, :] * shb
    dpd = jnp.dot(red_ref[...], prod)

    out_s = jnp.broadcast_to(gt[0:1, :], (16, be)) * ws
    out_p = jnp.zeros((16, be), jnp.float32)
    for u in range(3):
        out_p = out_p + (jnp.broadcast_to(dpd[u:u + 1, :], (16, be))
                         * wp[16 * u:16 * u + 16, :])
    out_d = jnp.zeros((16, be), jnp.float32)
    for u in range(5):
        out_d = out_d + (jnp.broadcast_to(dpd[3 + u:4 + u, :], (16, be))
                         * wd[16 * u:16 * u + 16, :])

    outt = jnp.concatenate([out_s, out_p, out_d], axis=0)
    out = outt.T
    s_ref[...] = jnp.concatenate(
        [out, jnp.zeros((be, 128 - 2 * SD), jnp.float32)], axis=1)


def _dense(G, vals, istep, a1, a2s, a2p, a2d):
    e = G.shape[0]
    be = 3200
    expm, red = _exp_red_mats()
    return pl.pallas_call(
        _dense_body,
        grid=(e // be,),
        in_specs=[
            pl.BlockSpec((be, 128), lambda i: (i, 0)),
            pl.BlockSpec((10, 1), lambda i: (0, 0)),
            pl.BlockSpec((1, 1), lambda i: (0, 0)),
            pl.BlockSpec((192, 10), lambda i: (0, 0)),
            pl.BlockSpec((16, 64), lambda i: (0, 0)),
            pl.BlockSpec((48, 64), lambda i: (0, 0)),
            pl.BlockSpec((80, 64), lambda i: (0, 0)),
            pl.BlockSpec((34, 8), lambda i: (0, 0)),
            pl.BlockSpec((8, 34), lambda i: (0, 0)),
        ],
        out_specs=[
            pl.BlockSpec((be, 128), lambda i: (i, 0)),
        ],
        out_shape=[
            jax.ShapeDtypeStruct((e, 128), jnp.float32),
        ],
    )(G, vals, istep, a1, a2s, a2p, a2d,
      jnp.asarray(expm), jnp.asarray(red))[0]


SCH = 5
SBROWS = 65


def _scatter_body(s_hbm, col2, z_hbm, o_hbm,
                  acc, cidx, rows_v, sem, sema,
                  n_per_s, n, srows, snx, sbrows, rofs):
    c = lax.axis_index("c")
    s = lax.axis_index("s")
    r0 = s * n_per_s
    n_last = n - (NS - 1) * n_per_s

    @pl.when(s < NS - 1)
    def _():
        pltpu.sync_copy(z_hbm.at[pl.ds(0, n_per_s)], acc.at[pl.ds(r0, n_per_s)])

    @pl.when(s == NS - 1)
    def _():
        pltpu.sync_copy(z_hbm.at[pl.ds(0, n_last)], acc.at[pl.ds(r0, n_last)])

    rbase = s * srows + jnp.minimum(s, snx)
    extra = s < snx
    snblk = srows // sbrows

    plsc.subcore_barrier()
    coff = c * SD

    def do_rows(gr0, lr0, k):
        e0 = (rbase + gr0) * 128
        pltpu.async_copy(s_hbm.at[pl.ds(e0, k * 128), pl.ds(coff, SD)],
                         rows_v.at[pl.ds(0, k * 128)], sem).wait()
        ds_ = []
        for j in range(k):
            ds_.append(pltpu.async_copy(rows_v.at[pl.ds(j * 128, 128)],
                                        acc.at[cidx.at[lr0 + j]], sema, add=True))
        for d in ds_:
            d.wait()

    def blk(b, carry):
        last = jnp.logical_and(b == snblk - 1, extra)

        @pl.when(last)
        def _():
            pltpu.sync_copy(col2.at[pl.ds(rofs + rbase + b * sbrows, sbrows + 1)],
                            cidx)

        @pl.when(jnp.logical_not(last))
        def _():
            pltpu.sync_copy(col2.at[pl.ds(rofs + rbase + b * sbrows, sbrows)],
                            cidx.at[pl.ds(0, sbrows)])

        def body(m, carry2):
            do_rows(b * sbrows + m * SCH, m * SCH, SCH)
            return carry2

        lax.fori_loop(0, sbrows // SCH, body, 0)

        @pl.when(last)
        def _():
            do_rows(srows, sbrows, 1)

        return carry

    lax.fori_loop(0, snblk, blk, 0)

    plsc.subcore_barrier()

    @pl.when(s < NS - 1)
    def _():
        pltpu.sync_copy(acc.at[pl.ds(r0, n_per_s)],
                        o_hbm.at[pl.ds(r0, n_per_s), pl.ds(coff, SD)])

    @pl.when(s == NS - 1)
    def _():
        pltpu.sync_copy(acc.at[pl.ds(r0, n_last)],
                        o_hbm.at[pl.ds(r0, n_last), pl.ds(coff, SD)])


def _scatter(S, col2, rofs, rows, n):
    srows, snx = rows // NS, rows % NS
    sbrows = srows // 2
    assert sbrows % SCH == 0 and srows == 2 * sbrows and sbrows <= 300
    n_per_s = -(-n // NS)
    n_per_s += (-n_per_s) % 8
    assert (NS - 1) * n_per_s < n

    z = jnp.zeros((n_per_s, SD), jnp.float32)
    mesh = plsc.VectorSubcoreMesh(core_axis_name="c", subcore_axis_name="s")
    scratch = [
        pltpu.VMEM_SHARED((n, SD), jnp.float32),
        pltpu.VMEM((sbrows + 1, 128), jnp.int32),
        pltpu.VMEM((SCH * 128, SD), jnp.float32),
        pltpu.SemaphoreType.DMA,
        pltpu.SemaphoreType.DMA,
    ]
    kfn = functools.partial(
        pl.kernel,
        mesh=mesh,
        out_type=jax.ShapeDtypeStruct((n, 128), jnp.float32),
        scratch_types=scratch,
        compiler_params=pltpu.CompilerParams(use_tc_tiling_on_sc=False),
    )(functools.partial(_scatter_body, n_per_s=n_per_s, n=n,
                        srows=srows, snx=snx, sbrows=sbrows, rofs=rofs))
    return kfn(S, col2, z)


def kernel(f_in, edge_index, pos, max_radius, num_nodes, W1s, W2s, W1p, W2p, W1d, W2d):
    n = f_in.shape[0]
    e = edge_index.shape[1]
    row2 = edge_index[0].reshape(-1, 128)
    col2 = edge_index[1].reshape(-1, 128)

    num_neighbors = e / n
    a1 = (jnp.concatenate([W1s, W1p, W1d], axis=1) * (1.0 / math.sqrt(10.0))).T
    cs = math.sqrt(2.0) / 8.0 / math.sqrt(num_neighbors)
    a2s = (W2s * cs).T
    a2p = (W2p * (cs / 3.0)).T
    a2d = (W2d * (cs / 5.0)).T

    mr = jnp.asarray(max_radius, jnp.float32)
    step = mr / (10 + 1)
    vals = (jnp.arange(1, 11, dtype=jnp.float32) * step).reshape(10, 1)
    istep = (1.0 / step).reshape(1, 1)

    P, PT = _pack(f_in, pos)
    rows = row2.shape[0]
    bounds = [0, 1600, 3200, 4800, rows]
    O = None
    for h in range(len(bounds) - 1):
        rofs, rcnt = bounds[h], bounds[h + 1] - bounds[h]
        Gh = _gather(P, PT, row2, col2, rofs, rcnt)
        Sh = _dense(Gh, vals, istep, a1, a2s, a2p, a2d)
        Oh = _scatter(Sh, col2, rofs, rcnt, n)
        O = Oh if O is None else O + Oh
    return O[:, 0:2 * SD]


# speedup vs baseline: 26.1217x; 1.0104x over previous
import functools
import math

import jax
import jax.numpy as jnp
import numpy as np
from jax import lax
from jax.experimental import pallas as pl
from jax.experimental.pallas import tpu as pltpu
from jax.experimental.pallas import tpu_sc as plsc

NC = 2
NS = 16
CH = 128

PD = 48
CD = 16
SD = 24


def _sel_matrix():
    srcs = ([0]
            + [(1 + u) * 9 + (1 + m) for u in range(3) for m in range(3)]
            + [(4 + u) * 9 + (4 + m) for u in range(5) for m in range(5)])
    sel = np.zeros((162, 35), np.float32)
    for j, k in enumerate(srcs):
        sel[2 * k, j] = 1.0
        sel[2 * k + 1, j] = 1.0
    return sel


def _pack_body(f_ref, pos_ref, sel_ref, p_ref, pt_ref):
    f = f_ref[...]
    bn = f.shape[0]
    pcols = jnp.dot(f, sel_ref[...], precision=lax.Precision.HIGHEST)
    posb = pos_ref[...]
    zp = jnp.zeros((bn, PD - 38), jnp.float32)
    p_ref[...] = jnp.concatenate([pcols, posb, zp], axis=1)
    pt_ref[...] = jnp.concatenate([posb, jnp.zeros((bn, CD - 3), jnp.float32)], axis=1)


def _pack(f_in, pos):
    n = f_in.shape[0]
    bn = 2000
    return pl.pallas_call(
        _pack_body,
        grid=(n // bn,),
        in_specs=[
            pl.BlockSpec((bn, 162), lambda i: (i, 0)),
            pl.BlockSpec((bn, 3), lambda i: (i, 0)),
            pl.BlockSpec((162, 35), lambda i: (0, 0)),
        ],
        out_specs=[
            pl.BlockSpec((bn, PD), lambda i: (i, 0)),
            pl.BlockSpec((bn, CD), lambda i: (i, 0)),
        ],
        out_shape=[
            jax.ShapeDtypeStruct((n, PD), jnp.float32),
            jax.ShapeDtypeStruct((n, CD), jnp.float32),
        ],
    )(f_in, pos, jnp.asarray(_sel_matrix()))


GCH = 5


def _gather_body(p_hbm, pt_hbm, row2, col2, g_hbm,
                 ridx, cidx, rows_v, crows_v, sem, semw, grows, nx, rofs):
    w = lax.axis_index("c") * NS + lax.axis_index("s")
    rbase = w * grows + jnp.minimum(w, nx)
    hbase = rofs + rbase
    extra = w < nx

    @pl.when(extra)
    def _():
        pltpu.sync_copy(row2.at[pl.ds(hbase, grows + 1)], ridx)
        pltpu.sync_copy(col2.at[pl.ds(hbase, grows + 1)], cidx)

    @pl.when(jnp.logical_not(extra))
    def _():
        pltpu.sync_copy(row2.at[pl.ds(hbase, grows)], ridx.at[pl.ds(0, grows)])
        pltpu.sync_copy(col2.at[pl.ds(hbase, grows)], cidx.at[pl.ds(0, grows)])

    def do_rows(r0, k):
        ds_ = []
        for j in range(k):
            ds_.append(pltpu.async_copy(p_hbm.at[ridx.at[r0 + j]],
                                        rows_v.at[pl.ds(j * 128, 128)], sem))
            ds_.append(pltpu.async_copy(pt_hbm.at[cidx.at[r0 + j]],
                                        crows_v.at[pl.ds(j * 128, 128)], sem))
        for d in ds_:
            d.wait()
        e0 = (rbase + r0) * 128
        d1 = pltpu.async_copy(rows_v.at[pl.ds(0, k * 128)],
                              g_hbm.at[pl.ds(e0, k * 128), pl.ds(0, PD)], semw)
        d2 = pltpu.async_copy(crows_v.at[pl.ds(0, k * 128)],
                              g_hbm.at[pl.ds(e0, k * 128), pl.ds(PD, CD)], semw)
        d1.wait()
        d2.wait()

    def body(m, carry):
        do_rows(m * GCH, GCH)
        return carry

    lax.fori_loop(0, grows // GCH, body, 0)
    for r in range(grows % GCH):
        do_rows((grows // GCH) * GCH + r, 1)

    @pl.when(extra)
    def _():
        do_rows(grows, 1)


def _gather(P, PT, row2, col2, rofs, rows):
    e = rows * row2.shape[1]
    nw = NC * NS
    grows, nx = rows // nw, rows % nw
    assert grows % GCH == 0
    mesh = plsc.VectorSubcoreMesh(core_axis_name="c", subcore_axis_name="s")
    scratch = [
        pltpu.VMEM((grows + 1, 128), jnp.int32),
        pltpu.VMEM((grows + 1, 128), jnp.int32),
        pltpu.VMEM((GCH * 128, PD), jnp.float32),
        pltpu.VMEM((GCH * 128, CD), jnp.float32),
        pltpu.SemaphoreType.DMA,
        pltpu.SemaphoreType.DMA,
    ]
    kfn = functools.partial(
        pl.kernel,
        mesh=mesh,
        out_type=jax.ShapeDtypeStruct((e, 128), jnp.float32),
        scratch_types=scratch,
        compiler_params=pltpu.CompilerParams(use_tc_tiling_on_sc=False),
    )(functools.partial(_gather_body, grows=grows, nx=nx, rofs=rofs))
    return kfn(P, PT, row2, col2)


def _exp_red_mats():
    expm = np.zeros((34, 8), np.float32)
    red = np.zeros((8, 34), np.float32)
    for r in range(9):
        expm[r, r % 3] = 1.0
        red[r // 3, r] = 1.0
    for r in range(9, 34):
        expm[r, 3 + (r - 9) % 5] = 1.0
        red[3 + (r - 9) // 5, r] = 1.0
    return expm, red


def _dense_body(g_ref, vals_ref, istep_ref, a1_ref, a2s_ref, a2p_ref, a2d_ref,
                expm_ref, red_ref, s_ref):
    be = g_ref.shape[0]
    gt = g_ref[:, 0:64].T

    v = gt[35:38, :] - gt[PD:PD + 3, :]
    n2 = jnp.sum(v * v, axis=0, keepdims=True) + 1e-12
    ln = jnp.sqrt(n2)
    inv = 1.0 / ln
    x, y, z = v[0:1, :] * inv, v[1:2, :] * inv, v[2:3, :] * inv
    s3 = math.sqrt(3.0)
    s15 = math.sqrt(15.0)
    shvec = jnp.concatenate([
        s3 * x, s3 * y, s3 * z,
        s15 * x * y, s15 * y * z, (math.sqrt(5.0) / 2.0) * (3.0 * z * z - 1.0),
        s15 * x * z, (s15 / 2.0) * (x * x - y * y)], axis=0)

    lnb = jnp.broadcast_to(ln, (10, be))
    diff = (lnb - vals_ref[...]) * istep_ref[...]
    def sus(t):
        safe = jnp.where(t > 0.0, t, 1.0)
        return jnp.where(t > 0.0, jnp.exp(-1.0 / safe), 0.0)
    cemb = 1.14136 * math.exp(2.0) * math.sqrt(10.0)
    emb = cemb * sus(diff + 1.0) * sus(1.0 - diff)

    h = jnp.maximum(jnp.dot(a1_ref[...], emb), 0.0)
    ws = jnp.dot(a2s_ref[...], h[0:64, :])
    wp = jnp.dot(a2p_ref[...], h[64:128, :])
    wd = jnp.dot(a2d_ref[...], h[128:192, :])

    shb = jnp.dot(expm_ref[...], shvec)
    prod = gt[1:35, :] * shb
    dpd = jnp.dot(red_ref[...], prod)

    out_s = jnp.broadcast_to(gt[0:1, :], (16, be)) * ws
    out_p = jnp.zeros((16, be), jnp.float32)
    for u in range(3):
        out_p = out_p + (jnp.broadcast_to(dpd[u:u + 1, :], (16, be))
                         * wp[16 * u:16 * u + 16, :])
    out_d = jnp.zeros((16, be), jnp.float32)
    for u in range(5):
        out_d = out_d + (jnp.broadcast_to(dpd[3 + u:4 + u, :], (16, be))
                         * wd[16 * u:16 * u + 16, :])

    outt = jnp.concatenate([out_s, out_p, out_d], axis=0)
    out = outt.T
    s_ref[...] = jnp.concatenate(
        [out, jnp.zeros((be, 128 - 2 * SD), jnp.float32)], axis=1)


def _dense(G, vals, istep, a1, a2s, a2p, a2d):
    e = G.shape[0]
    be = 3200
    expm, red = _exp_red_mats()
    return pl.pallas_call(
        _dense_body,
        grid=(e // be,),
        in_specs=[
            pl.BlockSpec((be, 128), lambda i: (i, 0)),
            pl.BlockSpec((10, 1), lambda i: (0, 0)),
            pl.BlockSpec((1, 1), lambda i: (0, 0)),
            pl.BlockSpec((192, 10), lambda i: (0, 0)),
            pl.BlockSpec((16, 64), lambda i: (0, 0)),
            pl.BlockSpec((48, 64), lambda i: (0, 0)),
            pl.BlockSpec((80, 64), lambda i: (0, 0)),
            pl.BlockSpec((34, 8), lambda i: (0, 0)),
            pl.BlockSpec((8, 34), lambda i: (0, 0)),
        ],
        out_specs=[
            pl.BlockSpec((be, 128), lambda i: (i, 0)),
        ],
        out_shape=[
            jax.ShapeDtypeStruct((e, 128), jnp.float32),
        ],
    )(G, vals, istep, a1, a2s, a2p, a2d,
      jnp.asarray(expm), jnp.asarray(red))[0]


SCH = 5
SBROWS = 65


def _scatter_body(s_hbm, col2, z_hbm, o_hbm,
                  acc, cidx, rows_v, sem, sema,
                  n_per_s, n, srows, snx, sbrows, rofs, chained):
    c = lax.axis_index("c")
    s = lax.axis_index("s")
    coff0 = c * SD
    r0 = s * n_per_s
    n_last = n - (NS - 1) * n_per_s

    def init(nr):
        if chained:
            pltpu.sync_copy(z_hbm.at[pl.ds(r0, nr), pl.ds(coff0, SD)],
                            acc.at[pl.ds(r0, nr)])
        else:
            pltpu.sync_copy(z_hbm.at[pl.ds(0, nr)], acc.at[pl.ds(r0, nr)])

    @pl.when(s < NS - 1)
    def _():
        init(n_per_s)

    @pl.when(s == NS - 1)
    def _():
        init(n_last)

    rbase = s * srows + jnp.minimum(s, snx)
    extra = s < snx
    snblk = srows // sbrows

    plsc.subcore_barrier()
    coff = c * SD

    def do_rows(gr0, lr0, k):
        e0 = (rbase + gr0) * 128
        pltpu.async_copy(s_hbm.at[pl.ds(e0, k * 128), pl.ds(coff, SD)],
                         rows_v.at[pl.ds(0, k * 128)], sem).wait()
        ds_ = []
        for j in range(k):
            ds_.append(pltpu.async_copy(rows_v.at[pl.ds(j * 128, 128)],
                                        acc.at[cidx.at[lr0 + j]], sema, add=True))
        for d in ds_:
            d.wait()

    def blk(b, carry):
        last = jnp.logical_and(b == snblk - 1, extra)

        @pl.when(last)
        def _():
            pltpu.sync_copy(col2.at[pl.ds(rofs + rbase + b * sbrows, sbrows + 1)],
                            cidx)

        @pl.when(jnp.logical_not(last))
        def _():
            pltpu.sync_copy(col2.at[pl.ds(rofs + rbase + b * sbrows, sbrows)],
                            cidx.at[pl.ds(0, sbrows)])

        def body(m, carry2):
            do_rows(b * sbrows + m * SCH, m * SCH, SCH)
            return carry2

        lax.fori_loop(0, sbrows // SCH, body, 0)

        @pl.when(last)
        def _():
            do_rows(srows, sbrows, 1)

        return carry

    lax.fori_loop(0, snblk, blk, 0)

    plsc.subcore_barrier()

    @pl.when(s < NS - 1)
    def _():
        pltpu.sync_copy(acc.at[pl.ds(r0, n_per_s)],
                        o_hbm.at[pl.ds(r0, n_per_s), pl.ds(coff, SD)])

    @pl.when(s == NS - 1)
    def _():
        pltpu.sync_copy(acc.at[pl.ds(r0, n_last)],
                        o_hbm.at[pl.ds(r0, n_last), pl.ds(coff, SD)])


def _scatter(S, col2, rofs, rows, n, o_prev=None):
    srows, snx = rows // NS, rows % NS
    sbrows = srows // 2
    assert sbrows % SCH == 0 and srows == 2 * sbrows and sbrows <= 300
    n_per_s = -(-n // NS)
    n_per_s += (-n_per_s) % 8
    assert (NS - 1) * n_per_s < n

    init = jnp.zeros((n_per_s, SD), jnp.float32) if o_prev is None else o_prev
    mesh = plsc.VectorSubcoreMesh(core_axis_name="c", subcore_axis_name="s")
    scratch = [
        pltpu.VMEM_SHARED((n, SD), jnp.float32),
        pltpu.VMEM((sbrows + 1, 128), jnp.int32),
        pltpu.VMEM((SCH * 128, SD), jnp.float32),
        pltpu.SemaphoreType.DMA,
        pltpu.SemaphoreType.DMA,
    ]
    kfn = functools.partial(
        pl.kernel,
        mesh=mesh,
        out_type=jax.ShapeDtypeStruct((n, 128), jnp.float32),
        scratch_types=scratch,
        compiler_params=pltpu.CompilerParams(use_tc_tiling_on_sc=False),
    )(functools.partial(_scatter_body, n_per_s=n_per_s, n=n,
                        srows=srows, snx=snx, sbrows=sbrows, rofs=rofs,
                        chained=o_prev is not None))
    return kfn(S, col2, init)


def kernel(f_in, edge_index, pos, max_radius, num_nodes, W1s, W2s, W1p, W2p, W1d, W2d):
    n = f_in.shape[0]
    e = edge_index.shape[1]
    row2 = edge_index[0].reshape(-1, 128)
    col2 = edge_index[1].reshape(-1, 128)

    num_neighbors = e / n
    a1 = (jnp.concatenate([W1s, W1p, W1d], axis=1) * (1.0 / math.sqrt(10.0))).T
    cs = math.sqrt(2.0) / 8.0 / math.sqrt(num_neighbors)
    a2s = (W2s * cs).T
    a2p = (W2p * (cs / 3.0)).T
    a2d = (W2d * (cs / 5.0)).T

    mr = jnp.asarray(max_radius, jnp.float32)
    step = mr / (10 + 1)
    vals = (jnp.arange(1, 11, dtype=jnp.float32) * step).reshape(10, 1)
    istep = (1.0 / step).reshape(1, 1)

    P, PT = _pack(f_in, pos)
    sizes = [800, 1600, 1925, 1125, 800]
    O = None
    rofs = 0
    for rcnt in sizes:
        Gh = _gather(P, PT, row2, col2, rofs, rcnt)
        Sh = _dense(Gh, vals, istep, a1, a2s, a2p, a2d)
        O = _scatter(Sh, col2, rofs, rcnt, n, o_prev=O)
        rofs += rcnt
    return O[:, 0:2 * SD]


# speedup vs baseline: 26.1369x; 1.0006x over previous
import functools
import math

import jax
import jax.numpy as jnp
import numpy as np
from jax import lax
from jax.experimental import pallas as pl
from jax.experimental.pallas import tpu as pltpu
from jax.experimental.pallas import tpu_sc as plsc

NC = 2
NS = 16
CH = 128

PD = 48
CD = 16
SD = 24


def _sel_matrix():
    srcs = ([0]
            + [(1 + u) * 9 + (1 + m) for u in range(3) for m in range(3)]
            + [(4 + u) * 9 + (4 + m) for u in range(5) for m in range(5)])
    sel = np.zeros((162, 35), np.float32)
    for j, k in enumerate(srcs):
        sel[2 * k, j] = 1.0
        sel[2 * k + 1, j] = 1.0
    return sel


def _pack_body(f_ref, pos_ref, sel_ref, p_ref, pt_ref):
    f = f_ref[...]
    bn = f.shape[0]
    pcols = jnp.dot(f, sel_ref[...], precision=lax.Precision.HIGHEST)
    posb = pos_ref[...]
    zp = jnp.zeros((bn, PD - 38), jnp.float32)
    p_ref[...] = jnp.concatenate([pcols, posb, zp], axis=1)
    pt_ref[...] = jnp.concatenate([posb, jnp.zeros((bn, CD - 3), jnp.float32)], axis=1)


def _pack(f_in, pos):
    n = f_in.shape[0]
    bn = 2000
    return pl.pallas_call(
        _pack_body,
        grid=(n // bn,),
        in_specs=[
            pl.BlockSpec((bn, 162), lambda i: (i, 0)),
            pl.BlockSpec((bn, 3), lambda i: (i, 0)),
            pl.BlockSpec((162, 35), lambda i: (0, 0)),
        ],
        out_specs=[
            pl.BlockSpec((bn, PD), lambda i: (i, 0)),
            pl.BlockSpec((bn, CD), lambda i: (i, 0)),
        ],
        out_shape=[
            jax.ShapeDtypeStruct((n, PD), jnp.float32),
            jax.ShapeDtypeStruct((n, CD), jnp.float32),
        ],
    )(f_in, pos, jnp.asarray(_sel_matrix()))


GCH = 5


def _gather_body(p_hbm, pt_hbm, row2, col2, g_hbm,
                 ridx, cidx, rows_v, crows_v, sem, semw, grows, nx, rofs):
    w = lax.axis_index("c") * NS + lax.axis_index("s")
    rbase = w * grows + jnp.minimum(w, nx)
    hbase = rofs + rbase
    extra = w < nx

    @pl.when(extra)
    def _():
        pltpu.sync_copy(row2.at[pl.ds(hbase, grows + 1)], ridx)
        pltpu.sync_copy(col2.at[pl.ds(hbase, grows + 1)], cidx)

    @pl.when(jnp.logical_not(extra))
    def _():
        pltpu.sync_copy(row2.at[pl.ds(hbase, grows)], ridx.at[pl.ds(0, grows)])
        pltpu.sync_copy(col2.at[pl.ds(hbase, grows)], cidx.at[pl.ds(0, grows)])

    def do_rows(r0, k):
        ds_ = []
        for j in range(k):
            ds_.append(pltpu.async_copy(p_hbm.at[ridx.at[r0 + j]],
                                        rows_v.at[pl.ds(j * 128, 128)], sem))
            ds_.append(pltpu.async_copy(pt_hbm.at[cidx.at[r0 + j]],
                                        crows_v.at[pl.ds(j * 128, 128)], sem))
        for d in ds_:
            d.wait()
        e0 = (rbase + r0) * 128
        d1 = pltpu.async_copy(rows_v.at[pl.ds(0, k * 128)],
                              g_hbm.at[pl.ds(e0, k * 128), pl.ds(0, PD)], semw)
        d2 = pltpu.async_copy(crows_v.at[pl.ds(0, k * 128)],
                              g_hbm.at[pl.ds(e0, k * 128), pl.ds(PD, CD)], semw)
        d1.wait()
        d2.wait()

    def body(m, carry):
        do_rows(m * GCH, GCH)
        return carry

    lax.fori_loop(0, grows // GCH, body, 0)
    for r in range(grows % GCH):
        do_rows((grows // GCH) * GCH + r, 1)

    @pl.when(extra)
    def _():
        do_rows(grows, 1)


def _gather(P, PT, row2, col2, rofs, rows):
    e = rows * row2.shape[1]
    nw = NC * NS
    grows, nx = rows // nw, rows % nw
    assert grows % GCH == 0
    mesh = plsc.VectorSubcoreMesh(core_axis_name="c", subcore_axis_name="s")
    scratch = [
        pltpu.VMEM((grows + 1, 128), jnp.int32),
        pltpu.VMEM((grows + 1, 128), jnp.int32),
        pltpu.VMEM((GCH * 128, PD), jnp.float32),
        pltpu.VMEM((GCH * 128, CD), jnp.float32),
        pltpu.SemaphoreType.DMA,
        pltpu.SemaphoreType.DMA,
    ]
    kfn = functools.partial(
        pl.kernel,
        mesh=mesh,
        out_type=jax.ShapeDtypeStruct((e, 128), jnp.float32),
        scratch_types=scratch,
        compiler_params=pltpu.CompilerParams(use_tc_tiling_on_sc=False),
    )(functools.partial(_gather_body, grows=grows, nx=nx, rofs=rofs))
    return kfn(P, PT, row2, col2)


def _exp_red_mats():
    expm = np.zeros((34, 8), np.float32)
    red = np.zeros((8, 34), np.float32)
    for r in range(9):
        expm[r, r % 3] = 1.0
        red[r // 3, r] = 1.0
    for r in range(9, 34):
        expm[r, 3 + (r - 9) % 5] = 1.0
        red[3 + (r - 9) // 5, r] = 1.0
    return expm, red


def _dense_body(g_ref, vals_ref, istep_ref, a1_ref, a2s_ref, a2p_ref, a2d_ref,
                expm_ref, red_ref, s_ref):
    be = g_ref.shape[0]
    gt = g_ref[:, 0:64].T

    v = gt[35:38, :] - gt[PD:PD + 3, :]
    n2 = jnp.sum(v * v, axis=0, keepdims=True) + 1e-12
    ln = jnp.sqrt(n2)
    inv = 1.0 / ln
    x, y, z = v[0:1, :] * inv, v[1:2, :] * inv, v[2:3, :] * inv
    s3 = math.sqrt(3.0)
    s15 = math.sqrt(15.0)
    shvec = jnp.concatenate([
        s3 * x, s3 * y, s3 * z,
        s15 * x * y, s15 * y * z, (math.sqrt(5.0) / 2.0) * (3.0 * z * z - 1.0),
        s15 * x * z, (s15 / 2.0) * (x * x - y * y)], axis=0)

    lnb = jnp.broadcast_to(ln, (10, be))
    diff = (lnb - vals_ref[...]) * istep_ref[...]
    def sus(t):
        safe = jnp.where(t > 0.0, t, 1.0)
        return jnp.where(t > 0.0, jnp.exp(-1.0 / safe), 0.0)
    cemb = 1.14136 * math.exp(2.0) * math.sqrt(10.0)
    emb = cemb * sus(diff + 1.0) * sus(1.0 - diff)

    embh = emb.astype(jnp.bfloat16)
    h = jnp.maximum(jnp.dot(a1_ref[...], embh,
                            preferred_element_type=jnp.float32), 0.0)
    hh = h.astype(jnp.bfloat16)
    ws = jnp.dot(a2s_ref[...], hh[0:64, :],
                 preferred_element_type=jnp.float32)
    wp = jnp.dot(a2p_ref[...], hh[64:128, :],
                 preferred_element_type=jnp.float32)
    wd = jnp.dot(a2d_ref[...], hh[128:192, :],
                 preferred_element_type=jnp.float32)

    shb = jnp.dot(expm_ref[...], shvec)
    prod = gt[1:35, :] * shb
    dpd = jnp.dot(red_ref[...], prod)

    out_s = jnp.broadcast_to(gt[0:1, :], (16, be)) * ws
    out_p = jnp.zeros((16, be), jnp.float32)
    for u in range(3):
        out_p = out_p + (jnp.broadcast_to(dpd[u:u + 1, :], (16, be))
                         * wp[16 * u:16 * u + 16, :])
    out_d = jnp.zeros((16, be), jnp.float32)
    for u in range(5):
        out_d = out_d + (jnp.broadcast_to(dpd[3 + u:4 + u, :], (16, be))
                         * wd[16 * u:16 * u + 16, :])

    outt = jnp.concatenate([out_s, out_p, out_d], axis=0)
    out = outt.T
    s_ref[...] = jnp.concatenate(
        [out, jnp.zeros((be, 128 - 2 * SD), jnp.float32)], axis=1)


def _dense(G, vals, istep, a1, a2s, a2p, a2d):
    e = G.shape[0]
    be = 3200
    expm, red = _exp_red_mats()
    return pl.pallas_call(
        _dense_body,
        grid=(e // be,),
        in_specs=[
            pl.BlockSpec((be, 128), lambda i: (i, 0)),
            pl.BlockSpec((10, 1), lambda i: (0, 0)),
            pl.BlockSpec((1, 1), lambda i: (0, 0)),
            pl.BlockSpec((192, 10), lambda i: (0, 0)),
            pl.BlockSpec((16, 64), lambda i: (0, 0)),
            pl.BlockSpec((48, 64), lambda i: (0, 0)),
            pl.BlockSpec((80, 64), lambda i: (0, 0)),
            pl.BlockSpec((34, 8), lambda i: (0, 0)),
            pl.BlockSpec((8, 34), lambda i: (0, 0)),
        ],
        out_specs=[
            pl.BlockSpec((be, 128), lambda i: (i, 0)),
        ],
        out_shape=[
            jax.ShapeDtypeStruct((e, 128), jnp.float32),
        ],
    )(G, vals, istep, a1, a2s, a2p, a2d,
      jnp.asarray(expm), jnp.asarray(red))[0]


SCH = 5
SBROWS = 65


def _scatter_body(s_hbm, col2, z_hbm, o_hbm,
                  acc, cidx, rows_v, sem, sema,
                  n_per_s, n, srows, snx, sbrows, rofs, chained):
    c = lax.axis_index("c")
    s = lax.axis_index("s")
    coff0 = c * SD
    r0 = s * n_per_s
    n_last = n - (NS - 1) * n_per_s

    def init(nr):
        if chained:
            pltpu.sync_copy(z_hbm.at[pl.ds(r0, nr), pl.ds(coff0, SD)],
                            acc.at[pl.ds(r0, nr)])
        else:
            pltpu.sync_copy(z_hbm.at[pl.ds(0, nr)], acc.at[pl.ds(r0, nr)])

    @pl.when(s < NS - 1)
    def _():
        init(n_per_s)

    @pl.when(s == NS - 1)
    def _():
        init(n_last)

    rbase = s * srows + jnp.minimum(s, snx)
    extra = s < snx
    snblk = srows // sbrows

    plsc.subcore_barrier()
    coff = c * SD

    def do_rows(gr0, lr0, k):
        e0 = (rbase + gr0) * 128
        pltpu.async_copy(s_hbm.at[pl.ds(e0, k * 128), pl.ds(coff, SD)],
                         rows_v.at[pl.ds(0, k * 128)], sem).wait()
        ds_ = []
        for j in range(k):
            ds_.append(pltpu.async_copy(rows_v.at[pl.ds(j * 128, 128)],
                                        acc.at[cidx.at[lr0 + j]], sema, add=True))
        for d in ds_:
            d.wait()

    def blk(b, carry):
        last = jnp.logical_and(b == snblk - 1, extra)

        @pl.when(last)
        def _():
            pltpu.sync_copy(col2.at[pl.ds(rofs + rbase + b * sbrows, sbrows + 1)],
                            cidx)

        @pl.when(jnp.logical_not(last))
        def _():
            pltpu.sync_copy(col2.at[pl.ds(rofs + rbase + b * sbrows, sbrows)],
                            cidx.at[pl.ds(0, sbrows)])

        def body(m, carry2):
            do_rows(b * sbrows + m * SCH, m * SCH, SCH)
            return carry2

        lax.fori_loop(0, sbrows // SCH, body, 0)

        @pl.when(last)
        def _():
            do_rows(srows, sbrows, 1)

        return carry

    lax.fori_loop(0, snblk, blk, 0)

    plsc.subcore_barrier()

    @pl.when(s < NS - 1)
    def _():
        pltpu.sync_copy(acc.at[pl.ds(r0, n_per_s)],
                        o_hbm.at[pl.ds(r0, n_per_s), pl.ds(coff, SD)])

    @pl.when(s == NS - 1)
    def _():
        pltpu.sync_copy(acc.at[pl.ds(r0, n_last)],
                        o_hbm.at[pl.ds(r0, n_last), pl.ds(coff, SD)])


def _scatter(S, col2, rofs, rows, n, o_prev=None):
    srows, snx = rows // NS, rows % NS
    sbrows = srows // 2
    assert sbrows % SCH == 0 and srows == 2 * sbrows and sbrows <= 300
    n_per_s = -(-n // NS)
    n_per_s += (-n_per_s) % 8
    assert (NS - 1) * n_per_s < n

    init = jnp.zeros((n_per_s, SD), jnp.float32) if o_prev is None else o_prev
    mesh = plsc.VectorSubcoreMesh(core_axis_name="c", subcore_axis_name="s")
    scratch = [
        pltpu.VMEM_SHARED((n, SD), jnp.float32),
        pltpu.VMEM((sbrows + 1, 128), jnp.int32),
        pltpu.VMEM((SCH * 128, SD), jnp.float32),
        pltpu.SemaphoreType.DMA,
        pltpu.SemaphoreType.DMA,
    ]
    kfn = functools.partial(
        pl.kernel,
        mesh=mesh,
        out_type=jax.ShapeDtypeStruct((n, 128), jnp.float32),
        scratch_types=scratch,
        compiler_params=pltpu.CompilerParams(use_tc_tiling_on_sc=False),
    )(functools.partial(_scatter_body, n_per_s=n_per_s, n=n,
                        srows=srows, snx=snx, sbrows=sbrows, rofs=rofs,
                        chained=o_prev is not None))
    return kfn(S, col2, init)


def kernel(f_in, edge_index, pos, max_radius, num_nodes, W1s, W2s, W1p, W2p, W1d, W2d):
    n = f_in.shape[0]
    e = edge_index.shape[1]
    row2 = edge_index[0].reshape(-1, 128)
    col2 = edge_index[1].reshape(-1, 128)

    num_neighbors = e / n
    a1 = (jnp.concatenate([W1s, W1p, W1d], axis=1)
          * (1.0 / math.sqrt(10.0))).T.astype(jnp.bfloat16)
    cs = math.sqrt(2.0) / 8.0 / math.sqrt(num_neighbors)
    a2s = (W2s * cs).T.astype(jnp.bfloat16)
    a2p = (W2p * (cs / 3.0)).T.astype(jnp.bfloat16)
    a2d = (W2d * (cs / 5.0)).T.astype(jnp.bfloat16)

    mr = jnp.asarray(max_radius, jnp.float32)
    step = mr / (10 + 1)
    vals = (jnp.arange(1, 11, dtype=jnp.float32) * step).reshape(10, 1)
    istep = (1.0 / step).reshape(1, 1)

    P, PT = _pack(f_in, pos)
    sizes = [800, 1600, 1925, 1125, 800]
    O = None
    rofs = 0
    for rcnt in sizes:
        Gh = _gather(P, PT, row2, col2, rofs, rcnt)
        Sh = _dense(Gh, vals, istep, a1, a2s, a2p, a2d)
        O = _scatter(Sh, col2, rofs, rcnt, n, o_prev=O)
        rofs += rcnt
    return O[:, 0:2 * SD]


# speedup vs baseline: 26.1686x; 1.0012x over previous
import functools
import math

import jax
import jax.numpy as jnp
import numpy as np
from jax import lax
from jax.experimental import pallas as pl
from jax.experimental.pallas import tpu as pltpu
from jax.experimental.pallas import tpu_sc as plsc

NC = 2
NS = 16
CH = 128

PD = 48
CD = 16
SD = 24


def _sel_matrix():
    srcs = ([0]
            + [(1 + u) * 9 + (1 + m) for u in range(3) for m in range(3)]
            + [(4 + u) * 9 + (4 + m) for u in range(5) for m in range(5)])
    sel = np.zeros((162, 35), np.float32)
    for j, k in enumerate(srcs):
        sel[2 * k, j] = 1.0
        sel[2 * k + 1, j] = 1.0
    return sel


def _pack_body(f_ref, pos_ref, sel_ref, p_ref, pt_ref):
    f = f_ref[...]
    bn = f.shape[0]
    pcols = jnp.dot(f, sel_ref[...], precision=lax.Precision.HIGHEST)
    posb = pos_ref[...]
    zp = jnp.zeros((bn, PD - 38), jnp.float32)
    p_ref[...] = jnp.concatenate([pcols, posb, zp], axis=1)
    pt_ref[...] = jnp.concatenate([posb, jnp.zeros((bn, CD - 3), jnp.float32)], axis=1)


def _pack(f_in, pos):
    n = f_in.shape[0]
    bn = 2000
    return pl.pallas_call(
        _pack_body,
        grid=(n // bn,),
        in_specs=[
            pl.BlockSpec((bn, 162), lambda i: (i, 0)),
            pl.BlockSpec((bn, 3), lambda i: (i, 0)),
            pl.BlockSpec((162, 35), lambda i: (0, 0)),
        ],
        out_specs=[
            pl.BlockSpec((bn, PD), lambda i: (i, 0)),
            pl.BlockSpec((bn, CD), lambda i: (i, 0)),
        ],
        out_shape=[
            jax.ShapeDtypeStruct((n, PD), jnp.float32),
            jax.ShapeDtypeStruct((n, CD), jnp.float32),
        ],
    )(f_in, pos, jnp.asarray(_sel_matrix()))


GCH = 5


def _gather_body(p_hbm, pt_hbm, row2, col2, g_hbm,
                 ridx, cidx, rows_v, crows_v, sem, semw, grows, nx, rofs):
    w = lax.axis_index("c") * NS + lax.axis_index("s")
    rbase = w * grows + jnp.minimum(w, nx)
    hbase = rofs + rbase
    extra = w < nx

    @pl.when(extra)
    def _():
        pltpu.sync_copy(row2.at[pl.ds(hbase, grows + 1)], ridx)
        pltpu.sync_copy(col2.at[pl.ds(hbase, grows + 1)], cidx)

    @pl.when(jnp.logical_not(extra))
    def _():
        pltpu.sync_copy(row2.at[pl.ds(hbase, grows)], ridx.at[pl.ds(0, grows)])
        pltpu.sync_copy(col2.at[pl.ds(hbase, grows)], cidx.at[pl.ds(0, grows)])

    def do_rows(r0, k):
        ds_ = []
        for j in range(k):
            ds_.append(pltpu.async_copy(p_hbm.at[ridx.at[r0 + j]],
                                        rows_v.at[pl.ds(j * 128, 128)], sem))
            ds_.append(pltpu.async_copy(pt_hbm.at[cidx.at[r0 + j]],
                                        crows_v.at[pl.ds(j * 128, 128)], sem))
        for d in ds_:
            d.wait()
        e0 = (rbase + r0) * 128
        d1 = pltpu.async_copy(rows_v.at[pl.ds(0, k * 128)],
                              g_hbm.at[pl.ds(e0, k * 128), pl.ds(0, PD)], semw)
        d2 = pltpu.async_copy(crows_v.at[pl.ds(0, k * 128)],
                              g_hbm.at[pl.ds(e0, k * 128), pl.ds(PD, CD)], semw)
        d1.wait()
        d2.wait()

    def body(m, carry):
        do_rows(m * GCH, GCH)
        return carry

    lax.fori_loop(0, grows // GCH, body, 0)
    for r in range(grows % GCH):
        do_rows((grows // GCH) * GCH + r, 1)

    @pl.when(extra)
    def _():
        do_rows(grows, 1)


def _gather(P, PT, row2, col2, rofs, rows):
    e = rows * row2.shape[1]
    nw = NC * NS
    grows, nx = rows // nw, rows % nw
    assert grows % GCH == 0
    mesh = plsc.VectorSubcoreMesh(core_axis_name="c", subcore_axis_name="s")
    scratch = [
        pltpu.VMEM((grows + 1, 128), jnp.int32),
        pltpu.VMEM((grows + 1, 128), jnp.int32),
        pltpu.VMEM((GCH * 128, PD), jnp.float32),
        pltpu.VMEM((GCH * 128, CD), jnp.float32),
        pltpu.SemaphoreType.DMA,
        pltpu.SemaphoreType.DMA,
    ]
    kfn = functools.partial(
        pl.kernel,
        mesh=mesh,
        out_type=jax.ShapeDtypeStruct((e, 128), jnp.float32),
        scratch_types=scratch,
        compiler_params=pltpu.CompilerParams(use_tc_tiling_on_sc=False),
    )(functools.partial(_gather_body, grows=grows, nx=nx, rofs=rofs))
    return kfn(P, PT, row2, col2)


def _exp_red_mats():
    expm = np.zeros((34, 8), np.float32)
    red = np.zeros((8, 34), np.float32)
    for r in range(9):
        expm[r, r % 3] = 1.0
        red[r // 3, r] = 1.0
    for r in range(9, 34):
        expm[r, 3 + (r - 9) % 5] = 1.0
        red[3 + (r - 9) // 5, r] = 1.0
    return expm, red


def _dense_body(g_ref, vals_ref, istep_ref, a1_ref, a2s_ref, a2p_ref, a2d_ref,
                expm_ref, red_ref, s_ref):
    be = g_ref.shape[0]
    gt = g_ref[:, 0:64].T

    v = gt[35:38, :] - gt[PD:PD + 3, :]
    n2 = jnp.sum(v * v, axis=0, keepdims=True) + 1e-12
    ln = jnp.sqrt(n2)
    inv = 1.0 / ln
    x, y, z = v[0:1, :] * inv, v[1:2, :] * inv, v[2:3, :] * inv
    s3 = math.sqrt(3.0)
    s15 = math.sqrt(15.0)
    shvec = jnp.concatenate([
        s3 * x, s3 * y, s3 * z,
        s15 * x * y, s15 * y * z, (math.sqrt(5.0) / 2.0) * (3.0 * z * z - 1.0),
        s15 * x * z, (s15 / 2.0) * (x * x - y * y)], axis=0)

    lnb = jnp.broadcast_to(ln, (10, be))
    diff = (lnb - vals_ref[...]) * istep_ref[...]
    def sus(t):
        safe = jnp.where(t > 0.0, t, 1.0)
        return jnp.where(t > 0.0, jnp.exp(-1.0 / safe), 0.0)
    cemb = 1.14136 * math.exp(2.0) * math.sqrt(10.0)
    emb = cemb * sus(diff + 1.0) * sus(1.0 - diff)

    h = jnp.maximum(jnp.dot(a1_ref[...], emb), 0.0)
    ws = jnp.dot(a2s_ref[...], h[0:64, :])
    wp = jnp.dot(a2p_ref[...], h[64:128, :])
    wd = jnp.dot(a2d_ref[...], h[128:192, :])

    shb = jnp.dot(expm_ref[...], shvec)
    prod = gt[1:35, :] * shb
    dpd = jnp.dot(red_ref[...], prod)

    out_s = jnp.broadcast_to(gt[0:1, :], (16, be)) * ws
    out_p = jnp.zeros((16, be), jnp.float32)
    for u in range(3):
        out_p = out_p + (jnp.broadcast_to(dpd[u:u + 1, :], (16, be))
                         * wp[16 * u:16 * u + 16, :])
    out_d = jnp.zeros((16, be), jnp.float32)
    for u in range(5):
        out_d = out_d + (jnp.broadcast_to(dpd[3 + u:4 + u, :], (16, be))
                         * wd[16 * u:16 * u + 16, :])

    outt = jnp.concatenate([out_s, out_p, out_d], axis=0)
    out = outt.T
    s_ref[...] = jnp.concatenate(
        [out, jnp.zeros((be, 128 - 2 * SD), jnp.float32)], axis=1)


def _dense(G, vals, istep, a1, a2s, a2p, a2d):
    e = G.shape[0]
    be = 3200
    expm, red = _exp_red_mats()
    return pl.pallas_call(
        _dense_body,
        grid=(e // be,),
        in_specs=[
            pl.BlockSpec((be, 128), lambda i: (i, 0)),
            pl.BlockSpec((10, 1), lambda i: (0, 0)),
            pl.BlockSpec((1, 1), lambda i: (0, 0)),
            pl.BlockSpec((192, 10), lambda i: (0, 0)),
            pl.BlockSpec((16, 64), lambda i: (0, 0)),
            pl.BlockSpec((48, 64), lambda i: (0, 0)),
            pl.BlockSpec((80, 64), lambda i: (0, 0)),
            pl.BlockSpec((34, 8), lambda i: (0, 0)),
            pl.BlockSpec((8, 34), lambda i: (0, 0)),
        ],
        out_specs=[
            pl.BlockSpec((be, 128), lambda i: (i, 0)),
        ],
        out_shape=[
            jax.ShapeDtypeStruct((e, 128), jnp.float32),
        ],
    )(G, vals, istep, a1, a2s, a2p, a2d,
      jnp.asarray(expm), jnp.asarray(red))[0]


SCH = 5
SBROWS = 65


def _scatter_body(s_hbm, col2, z_hbm, o_hbm,
                  acc, cidx, rows_v, sem, sema,
                  n_per_s, n, srows, snx, sbrows, rofs, chained):
    c = lax.axis_index("c")
    s = lax.axis_index("s")
    coff0 = c * SD
    r0 = s * n_per_s
    n_last = n - (NS - 1) * n_per_s

    def init(nr):
        if chained:
            pltpu.sync_copy(z_hbm.at[pl.ds(r0, nr), pl.ds(coff0, SD)],
                            acc.at[pl.ds(r0, nr)])
        else:
            pltpu.sync_copy(z_hbm.at[pl.ds(0, nr)], acc.at[pl.ds(r0, nr)])

    @pl.when(s < NS - 1)
    def _():
        init(n_per_s)

    @pl.when(s == NS - 1)
    def _():
        init(n_last)

    rbase = s * srows + jnp.minimum(s, snx)
    extra = s < snx
    snblk = srows // sbrows

    plsc.subcore_barrier()
    coff = c * SD

    def do_rows(gr0, lr0, k):
        e0 = (rbase + gr0) * 128
        pltpu.async_copy(s_hbm.at[pl.ds(e0, k * 128), pl.ds(coff, SD)],
                         rows_v.at[pl.ds(0, k * 128)], sem).wait()
        ds_ = []
        for j in range(k):
            ds_.append(pltpu.async_copy(rows_v.at[pl.ds(j * 128, 128)],
                                        acc.at[cidx.at[lr0 + j]], sema, add=True))
        for d in ds_:
            d.wait()

    def blk(b, carry):
        last = jnp.logical_and(b == snblk - 1, extra)

        @pl.when(last)
        def _():
            pltpu.sync_copy(col2.at[pl.ds(rofs + rbase + b * sbrows, sbrows + 1)],
                            cidx)

        @pl.when(jnp.logical_not(last))
        def _():
            pltpu.sync_copy(col2.at[pl.ds(rofs + rbase + b * sbrows, sbrows)],
                            cidx.at[pl.ds(0, sbrows)])

        def body(m, carry2):
            do_rows(b * sbrows + m * SCH, m * SCH, SCH)
            return carry2

        lax.fori_loop(0, sbrows // SCH, body, 0)

        @pl.when(last)
        def _():
            do_rows(srows, sbrows, 1)

        return carry

    lax.fori_loop(0, snblk, blk, 0)

    plsc.subcore_barrier()

    @pl.when(s < NS - 1)
    def _():
        pltpu.sync_copy(acc.at[pl.ds(r0, n_per_s)],
                        o_hbm.at[pl.ds(r0, n_per_s), pl.ds(coff, SD)])

    @pl.when(s == NS - 1)
    def _():
        pltpu.sync_copy(acc.at[pl.ds(r0, n_last)],
                        o_hbm.at[pl.ds(r0, n_last), pl.ds(coff, SD)])


def _scatter(S, col2, rofs, rows, n, o_prev=None):
    srows, snx = rows // NS, rows % NS
    sbrows = srows // 2
    assert sbrows % SCH == 0 and srows == 2 * sbrows and sbrows <= 300
    n_per_s = -(-n // NS)
    n_per_s += (-n_per_s) % 8
    assert (NS - 1) * n_per_s < n

    init = jnp.zeros((n_per_s, SD), jnp.float32) if o_prev is None else o_prev
    mesh = plsc.VectorSubcoreMesh(core_axis_name="c", subcore_axis_name="s")
    scratch = [
        pltpu.VMEM_SHARED((n, SD), jnp.float32),
        pltpu.VMEM((sbrows + 1, 128), jnp.int32),
        pltpu.VMEM((SCH * 128, SD), jnp.float32),
        pltpu.SemaphoreType.DMA,
        pltpu.SemaphoreType.DMA,
    ]
    kfn = functools.partial(
        pl.kernel,
        mesh=mesh,
        out_type=jax.ShapeDtypeStruct((n, 128), jnp.float32),
        scratch_types=scratch,
        compiler_params=pltpu.CompilerParams(use_tc_tiling_on_sc=False),
    )(functools.partial(_scatter_body, n_per_s=n_per_s, n=n,
                        srows=srows, snx=snx, sbrows=sbrows, rofs=rofs,
                        chained=o_prev is not None))
    return kfn(S, col2, init)


def kernel(f_in, edge_index, pos, max_radius, num_nodes, W1s, W2s, W1p, W2p, W1d, W2d):
    n = f_in.shape[0]
    e = edge_index.shape[1]
    row2 = edge_index[0].reshape(-1, 128)
    col2 = edge_index[1].reshape(-1, 128)

    num_neighbors = e / n
    a1 = (jnp.concatenate([W1s, W1p, W1d], axis=1) * (1.0 / math.sqrt(10.0))).T
    cs = math.sqrt(2.0) / 8.0 / math.sqrt(num_neighbors)
    a2s = (W2s * cs).T
    a2p = (W2p * (cs / 3.0)).T
    a2d = (W2d * (cs / 5.0)).T

    mr = jnp.asarray(max_radius, jnp.float32)
    step = mr / (10 + 1)
    vals = (jnp.arange(1, 11, dtype=jnp.float32) * step).reshape(10, 1)
    istep = (1.0 / step).reshape(1, 1)

    P, PT = _pack(f_in, pos)
    sizes = [800, 1600, 1925, 1125, 800]
    O = None
    rofs = 0
    for rcnt in sizes:
        Gh = _gather(P, PT, row2, col2, rofs, rcnt)
        Sh = _dense(Gh, vals, istep, a1, a2s, a2p, a2d)
        O = _scatter(Sh, col2, rofs, rcnt, n, o_prev=O)
        rofs += rcnt
    return O[:, 0:2 * SD]
